# Initial kernel scaffold; baseline (speedup 1.0000x reference)
import jax
import jax.numpy as jnp
from jax.experimental import pallas as pl


def kernel(target_node_indices, emb, W_pre, b_pre, ln1_g, ln1_b, Wr, attn_vec, Ws, bs, bias, lnf_g, lnf_b, rows, cols, vals):
    raise NotImplementedError("write your pallas kernel here")


# speedup vs baseline: 2.3911x; 2.3911x over previous
import functools

import jax
import jax.numpy as jnp
import numpy as np
from jax import lax
from jax.experimental import pallas as pl
from jax.experimental.pallas import tpu as pltpu
from jax.experimental.pallas import tpu_sc as plsc

_N = 10000
_D = 128
_R = 3
_E = 160000

_NC = 2
_NT = 16
_K = 256
_EPT = 5120
_EPAD = _EPT * _NC * _NT
_NCHUNK = _EPT // _K
_RPT = 640
_SQRT1_2 = np.float32(0.7071067811865476)


def _gelu(h):
    return 0.5 * h * (1.0 + lax.erf(h * _SQRT1_2))


def _layernorm_in(h, g, b):
    mu = jnp.mean(h, axis=1, keepdims=True)
    var = jnp.mean((h - mu) ** 2, axis=1, keepdims=True)
    return (h - mu) * lax.rsqrt(var + np.float32(1e-5)) * g + b


def _pre_body(emb_ref, wpre_ref, bpre_ref, g1_ref, b1_ref, wr_ref, ws_ref,
              bs_ref, xr0_ref, xr1_ref, xr2_ref, sl_ref):
    e = emb_ref[...]
    h = lax.dot_general(e, wpre_ref[...], (((1,), (1,)), ((), ())),
                        preferred_element_type=jnp.float32) + bpre_ref[...]
    h = _layernorm_in(h, g1_ref[...], b1_ref[...])
    x = _gelu(h)
    wr = wr_ref[...]
    xr0_ref[...] = lax.dot_general(x, wr[0], (((1,), (1,)), ((), ())),
                                   preferred_element_type=jnp.float32)
    xr1_ref[...] = lax.dot_general(x, wr[1], (((1,), (1,)), ((), ())),
                                   preferred_element_type=jnp.float32)
    xr2_ref[...] = lax.dot_general(x, wr[2], (((1,), (1,)), ((), ())),
                                   preferred_element_type=jnp.float32)
    sl_ref[...] = lax.dot_general(x, ws_ref[...], (((1,), (1,)), ((), ())),
                                  preferred_element_type=jnp.float32) + bs_ref[...]


def _pre_call(emb, wpre, bpre, g1, b1, wr, ws, bs2):
    B = 1000
    n_blk = _N // B
    full = lambda i: (0, 0)
    row_spec = pl.BlockSpec((B, _D), lambda i: (i, 0))
    return pl.pallas_call(
        _pre_body,
        grid=(n_blk,),
        in_specs=[
            row_spec,
            pl.BlockSpec((_D, _D), full),
            pl.BlockSpec((1, _D), full),
            pl.BlockSpec((1, _D), full),
            pl.BlockSpec((1, _D), full),
            pl.BlockSpec((_R, _D, _D), lambda i: (0, 0, 0)),
            pl.BlockSpec((_D, _D), full),
            pl.BlockSpec((1, _D), full),
        ],
        out_specs=[row_spec, row_spec, row_spec, row_spec],
        out_shape=[jax.ShapeDtypeStruct((_N, _D), jnp.float32)] * 4,
    )(emb, wpre, bpre, g1, b1, wr, ws, bs2)


def _spmm_body(xr0_hbm, xr1_hbm, xr2_hbm,
               rows0, rows1, rows2, cols0, cols1, cols2, vals0, vals1, vals2,
               out_hbm, buf, colbuf, rowbuf, valbuf, acc, gsem):
    c = lax.axis_index("c")
    s = lax.axis_index("s")
    xr_list = [xr0_hbm, xr1_hbm, xr2_hbm]
    rows_list = [rows0, rows1, rows2]
    cols_list = [cols0, cols1, cols2]
    vals_list = [vals0, vals1, vals2]
    row0 = s * 624

    for r in range(_R):
        @pl.loop(0, _K)
        def _zero(k):
            for t in range(_D // 16):
                buf[k, pl.ds(t * 16, 16)] = jnp.zeros((16,), jnp.float32)

        pltpu.sync_copy(buf.at[pl.ds(0, 256), :], acc.at[pl.ds(row0, 256), :])
        pltpu.sync_copy(buf.at[pl.ds(0, 256), :], acc.at[pl.ds(row0 + 256, 256), :])
        pltpu.sync_copy(buf.at[pl.ds(0, 112), :], acc.at[pl.ds(row0 + 512, 112), :])

        @pl.when(s == _NT - 1)
        def _zero_tail():
            pltpu.sync_copy(buf.at[pl.ds(0, 16), :], acc.at[pl.ds(9984, 16), :])

        plsc.subcore_barrier()

        base = c * (_NT * _EPT) + s * _EPT
        xr = xr_list[r]
        rows_hbm = rows_list[r]
        cols_hbm = cols_list[r]
        vals_hbm = vals_list[r]

        @pl.loop(0, _NCHUNK)
        def _chunk(i):
            off = base + i * _K
            pltpu.sync_copy(cols_hbm.at[pl.ds(off, _K)], colbuf)
            pltpu.sync_copy(rows_hbm.at[pl.ds(off, _K)], rowbuf)
            pltpu.sync_copy(vals_hbm.at[pl.ds(off, _K)], valbuf)
            pltpu.async_copy(xr.at[colbuf], buf, gsem).wait()

            @pl.loop(0, _K // 16)
            def _grp(g):
                v16 = valbuf[pl.ds(g * 16, 16)]
                for j in range(16):
                    vj = v16[j]
                    k = g * 16 + j
                    for t in range(_D // 16):
                        buf[k, pl.ds(t * 16, 16)] = buf[k, pl.ds(t * 16, 16)] * vj

            pltpu.sync_copy(buf, acc.at[rowbuf], add=True)

        plsc.subcore_barrier()
        pltpu.sync_copy(acc.at[pl.ds(row0, 624), :],
                        out_hbm.at[r, c, pl.ds(row0, 624), :])

        @pl.when(s == _NT - 1)
        def _wb_tail():
            pltpu.sync_copy(acc.at[pl.ds(9984, 16), :],
                            out_hbm.at[r, c, pl.ds(9984, 16), :])

        plsc.subcore_barrier()


def _spmm_call(xr0, xr1, xr2, rows_p, cols_p, vals_p):
    mesh = plsc.VectorSubcoreMesh(core_axis_name="c", subcore_axis_name="s")
    f = pl.kernel(
        _spmm_body,
        out_type=jax.ShapeDtypeStruct((_R, _NC, _N, _D), jnp.float32),
        mesh=mesh,
        scratch_types=[
            pltpu.VMEM((_K, _D), jnp.float32),
            pltpu.VMEM((_K,), jnp.int32),
            pltpu.VMEM((_K,), jnp.int32),
            pltpu.VMEM((_K,), jnp.float32),
            pltpu.VMEM_SHARED((_N, _D), jnp.float32),
            pltpu.SemaphoreType.DMA,
        ],
    )
    return f(xr0, xr1, xr2, rows_p[0], rows_p[1], rows_p[2],
             cols_p[0], cols_p[1], cols_p[2], vals_p[0], vals_p[1], vals_p[2])


def _post_body(part_ref, sl_ref, av_ref, bias_ref, gf_ref, bf_ref,
               final_ref, attn_ref):
    m0 = part_ref[0, 0] + part_ref[0, 1]
    m1 = part_ref[1, 0] + part_ref[1, 1]
    m2 = part_ref[2, 0] + part_ref[2, 1]
    av = av_ref[...]
    s0 = jnp.sum(m0 * av, axis=1, keepdims=True)
    s1 = jnp.sum(m1 * av, axis=1, keepdims=True)
    s2 = jnp.sum(m2 * av, axis=1, keepdims=True)
    k0 = jnp.any(m0 != 0, axis=1, keepdims=True)
    k1 = jnp.any(m1 != 0, axis=1, keepdims=True)
    k2 = jnp.any(m2 != 0, axis=1, keepdims=True)
    neg = np.float32(-1e30)
    s0 = jnp.where(k0, s0, neg)
    s1 = jnp.where(k1, s1, neg)
    s2 = jnp.where(k2, s2, neg)
    mx = jnp.maximum(jnp.maximum(s0, s1), s2)
    e0 = jnp.where(k0, jnp.exp(s0 - mx), 0.0)
    e1 = jnp.where(k1, jnp.exp(s1 - mx), 0.0)
    e2 = jnp.where(k2, jnp.exp(s2 - mx), 0.0)
    den = e0 + e1 + e2
    inv = jnp.where(den > 0, 1.0 / den, 0.0)
    a0 = e0 * inv
    a1 = e1 * inv
    a2 = e2 * inv
    h = m0 * a0 + m1 * a1 + m2 * a2 + bias_ref[...] + sl_ref[...]
    out = _gelu(h)
    final_ref[...] = _layernorm_in(out, gf_ref[...], bf_ref[...])
    attn_ref[...] = jnp.concatenate([a0, a1, a2], axis=1)


def _post_call(part, sl, av2, bias2, gf, bf):
    B = 1000
    n_blk = _N // B
    full = lambda i: (0, 0)
    row_spec = pl.BlockSpec((B, _D), lambda i: (i, 0))
    return pl.pallas_call(
        _post_body,
        grid=(n_blk,),
        in_specs=[
            pl.BlockSpec((_R, _NC, B, _D), lambda i: (0, 0, i, 0)),
            row_spec,
            pl.BlockSpec((1, _D), full),
            pl.BlockSpec((1, _D), full),
            pl.BlockSpec((1, _D), full),
            pl.BlockSpec((1, _D), full),
        ],
        out_specs=[row_spec, pl.BlockSpec((B, _R), lambda i: (i, 0))],
        out_shape=[jax.ShapeDtypeStruct((_N, _D), jnp.float32),
                   jax.ShapeDtypeStruct((_N, _R), jnp.float32)],
    )(part, sl, av2, bias2, gf, bf)


def _tgt_body(final_hbm, idx_hbm, out_hbm, idxv, rowsv, sem):
    wid = lax.axis_index("s") * _NC + lax.axis_index("c")
    base = wid * 32
    pltpu.sync_copy(idx_hbm.at[pl.ds(base, 32)], idxv)
    pltpu.async_copy(final_hbm.at[idxv], rowsv, sem).wait()
    pltpu.sync_copy(rowsv, out_hbm.at[pl.ds(base, 32)])


def _tgt_call(final_full, tgt):
    mesh = plsc.VectorSubcoreMesh(core_axis_name="c", subcore_axis_name="s")
    f = pl.kernel(
        _tgt_body,
        out_type=jax.ShapeDtypeStruct((1024, _D), jnp.float32),
        mesh=mesh,
        scratch_types=[
            pltpu.VMEM((32,), jnp.int32),
            pltpu.VMEM((32, _D), jnp.float32),
            pltpu.SemaphoreType.DMA,
        ],
    )
    return f(final_full, tgt)


def kernel(target_node_indices, emb, W_pre, b_pre, ln1_g, ln1_b, Wr, attn_vec,
           Ws, bs, bias, lnf_g, lnf_b, rows, cols, vals):
    bpre2 = b_pre.reshape(1, _D)
    g12 = ln1_g.reshape(1, _D)
    b12 = ln1_b.reshape(1, _D)
    bs2 = bs.reshape(1, _D)
    bias2 = bias.reshape(1, _D)
    gf2 = lnf_g.reshape(1, _D)
    bf2 = lnf_b.reshape(1, _D)
    av2 = attn_vec.reshape(1, _D)

    xr0, xr1, xr2, sl = _pre_call(emb, W_pre, bpre2, g12, b12, Wr, Ws, bs2)

    pad = _EPAD - _E
    rows_p = jnp.pad(rows, ((0, 0), (0, pad)))
    cols_p = jnp.pad(cols, ((0, 0), (0, pad)))
    vals_p = jnp.pad(vals, ((0, 0), (0, pad)))

    part = _spmm_call(xr0, xr1, xr2, rows_p, cols_p, vals_p)

    final_full, attn = _post_call(part, sl, av2, bias2, gf2, bf2)
    final = _tgt_call(final_full, target_node_indices)
    return final, attn, emb


# speedup vs baseline: 2.9090x; 1.2166x over previous
import functools

import jax
import jax.numpy as jnp
import numpy as np
from jax import lax
from jax.experimental import pallas as pl
from jax.experimental.pallas import tpu as pltpu
from jax.experimental.pallas import tpu_sc as plsc

_N = 10000
_D = 128
_R = 3
_E = 160000

_NC = 2
_NT = 16
_K = 160
_EPT = 5120
_EPAD = _EPT * _NC * _NT
_NCHUNK = _EPT // _K
_RPT = 640
_SQRT1_2 = np.float32(0.7071067811865476)


def _gelu(h):
    return 0.5 * h * (1.0 + lax.erf(h * _SQRT1_2))


def _layernorm_in(h, g, b):
    mu = jnp.mean(h, axis=1, keepdims=True)
    var = jnp.mean((h - mu) ** 2, axis=1, keepdims=True)
    return (h - mu) * lax.rsqrt(var + np.float32(1e-5)) * g + b


def _pre_body(emb_ref, wpre_ref, bpre_ref, g1_ref, b1_ref, wr_ref, ws_ref,
              bs_ref, xr0_ref, xr1_ref, xr2_ref, sl_ref):
    e = emb_ref[...]
    h = lax.dot_general(e, wpre_ref[...], (((1,), (1,)), ((), ())),
                        preferred_element_type=jnp.float32) + bpre_ref[...]
    h = _layernorm_in(h, g1_ref[...], b1_ref[...])
    x = _gelu(h)
    wr = wr_ref[...]
    xr0_ref[...] = lax.dot_general(x, wr[0], (((1,), (1,)), ((), ())),
                                   preferred_element_type=jnp.float32)
    xr1_ref[...] = lax.dot_general(x, wr[1], (((1,), (1,)), ((), ())),
                                   preferred_element_type=jnp.float32)
    xr2_ref[...] = lax.dot_general(x, wr[2], (((1,), (1,)), ((), ())),
                                   preferred_element_type=jnp.float32)
    sl_ref[...] = lax.dot_general(x, ws_ref[...], (((1,), (1,)), ((), ())),
                                  preferred_element_type=jnp.float32) + bs_ref[...]


def _pre_call(emb, wpre, bpre, g1, b1, wr, ws, bs2):
    B = 1000
    n_blk = _N // B
    full = lambda i: (0, 0)
    row_spec = pl.BlockSpec((B, _D), lambda i: (i, 0))
    return pl.pallas_call(
        _pre_body,
        grid=(n_blk,),
        in_specs=[
            row_spec,
            pl.BlockSpec((_D, _D), full),
            pl.BlockSpec((1, _D), full),
            pl.BlockSpec((1, _D), full),
            pl.BlockSpec((1, _D), full),
            pl.BlockSpec((_R, _D, _D), lambda i: (0, 0, 0)),
            pl.BlockSpec((_D, _D), full),
            pl.BlockSpec((1, _D), full),
        ],
        out_specs=[row_spec, row_spec, row_spec, row_spec],
        out_shape=[jax.ShapeDtypeStruct((_N, _D), jnp.float32)] * 4,
    )(emb, wpre, bpre, g1, b1, wr, ws, bs2)


def _scale_chunk(buf, valbuf):
    @pl.loop(0, _K // 16)
    def _grp(g):
        v16 = valbuf[pl.ds(g * 16, 16)]
        for j in range(16):
            vj = v16[j]
            k = g * 16 + j
            for t in range(_D // 16):
                buf[k, pl.ds(t * 16, 16)] = buf[k, pl.ds(t * 16, 16)] * vj


def _spmm_body(xr0_hbm, xr1_hbm, xr2_hbm,
               rows0, rows1, rows2, cols0, cols1, cols2, vals0, vals1, vals2,
               out_hbm, bufa, bufb, cola, colb, rowa, rowb, vala, valb,
               acc, gsema, gsemb):
    c = lax.axis_index("c")
    s = lax.axis_index("s")
    xr_list = [xr0_hbm, xr1_hbm, xr2_hbm]
    rows_list = [rows0, rows1, rows2]
    cols_list = [cols0, cols1, cols2]
    vals_list = [vals0, vals1, vals2]
    row0 = s * 624

    for r in range(_R):
        @pl.loop(0, _K)
        def _zero(k):
            for t in range(_D // 16):
                bufa[k, pl.ds(t * 16, 16)] = jnp.zeros((16,), jnp.float32)

        pltpu.sync_copy(bufa.at[pl.ds(0, 160), :], acc.at[pl.ds(row0, 160), :])
        pltpu.sync_copy(bufa.at[pl.ds(0, 160), :], acc.at[pl.ds(row0 + 160, 160), :])
        pltpu.sync_copy(bufa.at[pl.ds(0, 160), :], acc.at[pl.ds(row0 + 320, 160), :])
        pltpu.sync_copy(bufa.at[pl.ds(0, 144), :], acc.at[pl.ds(row0 + 480, 144), :])

        @pl.when(s == _NT - 1)
        def _zero_tail():
            pltpu.sync_copy(bufa.at[pl.ds(0, 16), :], acc.at[pl.ds(9984, 16), :])

        plsc.subcore_barrier()

        base = c * (_NT * _EPT) + s * _EPT
        xr = xr_list[r]
        rows_hbm = rows_list[r]
        cols_hbm = cols_list[r]
        vals_hbm = vals_list[r]

        def _fetch(i, colq, rowq, valq, bufq, semq):
            off = base + i * _K
            pltpu.sync_copy(cols_hbm.at[pl.ds(off, _K)], colq)
            pltpu.sync_copy(rows_hbm.at[pl.ds(off, _K)], rowq)
            pltpu.sync_copy(vals_hbm.at[pl.ds(off, _K)], valq)
            return pltpu.async_copy(xr.at[colq], bufq, semq)

        _fetch(0, cola, rowa, vala, bufa, gsema)

        @pl.loop(0, _NCHUNK // 2)
        def _pair(i):
            hb = _fetch(2 * i + 1, colb, rowb, valb, bufb, gsemb)
            pltpu.make_async_copy(xr.at[cola], bufa, gsema).wait()
            _scale_chunk(bufa, vala)
            pltpu.sync_copy(bufa, acc.at[rowa], add=True)

            @pl.when(i < _NCHUNK // 2 - 1)
            def _next_a():
                _fetch(2 * i + 2, cola, rowa, vala, bufa, gsema)

            hb.wait()
            _scale_chunk(bufb, valb)
            pltpu.sync_copy(bufb, acc.at[rowb], add=True)

        plsc.subcore_barrier()
        pltpu.sync_copy(acc.at[pl.ds(row0, 624), :],
                        out_hbm.at[r, c, pl.ds(row0, 624), :])

        @pl.when(s == _NT - 1)
        def _wb_tail():
            pltpu.sync_copy(acc.at[pl.ds(9984, 16), :],
                            out_hbm.at[r, c, pl.ds(9984, 16), :])

        plsc.subcore_barrier()


def _spmm_call(xr0, xr1, xr2, rows_p, cols_p, vals_p):
    mesh = plsc.VectorSubcoreMesh(core_axis_name="c", subcore_axis_name="s")
    f = pl.kernel(
        _spmm_body,
        out_type=jax.ShapeDtypeStruct((_R, _NC, _N, _D), jnp.float32),
        mesh=mesh,
        scratch_types=[
            pltpu.VMEM((_K, _D), jnp.float32),
            pltpu.VMEM((_K, _D), jnp.float32),
            pltpu.VMEM((_K,), jnp.int32),
            pltpu.VMEM((_K,), jnp.int32),
            pltpu.VMEM((_K,), jnp.int32),
            pltpu.VMEM((_K,), jnp.int32),
            pltpu.VMEM((_K,), jnp.float32),
            pltpu.VMEM((_K,), jnp.float32),
            pltpu.VMEM_SHARED((_N, _D), jnp.float32),
            pltpu.SemaphoreType.DMA,
            pltpu.SemaphoreType.DMA,
        ],
    )
    return f(xr0, xr1, xr2, rows_p[0], rows_p[1], rows_p[2],
             cols_p[0], cols_p[1], cols_p[2], vals_p[0], vals_p[1], vals_p[2])


def _post_body(part_ref, sl_ref, av_ref, bias_ref, gf_ref, bf_ref,
               final_ref, attn_ref):
    m0 = part_ref[0, 0] + part_ref[0, 1]
    m1 = part_ref[1, 0] + part_ref[1, 1]
    m2 = part_ref[2, 0] + part_ref[2, 1]
    av = av_ref[...]
    s0 = jnp.sum(m0 * av, axis=1, keepdims=True)
    s1 = jnp.sum(m1 * av, axis=1, keepdims=True)
    s2 = jnp.sum(m2 * av, axis=1, keepdims=True)
    k0 = jnp.any(m0 != 0, axis=1, keepdims=True)
    k1 = jnp.any(m1 != 0, axis=1, keepdims=True)
    k2 = jnp.any(m2 != 0, axis=1, keepdims=True)
    neg = np.float32(-1e30)
    s0 = jnp.where(k0, s0, neg)
    s1 = jnp.where(k1, s1, neg)
    s2 = jnp.where(k2, s2, neg)
    mx = jnp.maximum(jnp.maximum(s0, s1), s2)
    e0 = jnp.where(k0, jnp.exp(s0 - mx), 0.0)
    e1 = jnp.where(k1, jnp.exp(s1 - mx), 0.0)
    e2 = jnp.where(k2, jnp.exp(s2 - mx), 0.0)
    den = e0 + e1 + e2
    inv = jnp.where(den > 0, 1.0 / den, 0.0)
    a0 = e0 * inv
    a1 = e1 * inv
    a2 = e2 * inv
    h = m0 * a0 + m1 * a1 + m2 * a2 + bias_ref[...] + sl_ref[...]
    out = _gelu(h)
    final_ref[...] = _layernorm_in(out, gf_ref[...], bf_ref[...])
    attn_ref[...] = jnp.concatenate([a0, a1, a2], axis=1)


def _post_call(part, sl, av2, bias2, gf, bf):
    B = 1000
    n_blk = _N // B
    full = lambda i: (0, 0)
    row_spec = pl.BlockSpec((B, _D), lambda i: (i, 0))
    return pl.pallas_call(
        _post_body,
        grid=(n_blk,),
        in_specs=[
            pl.BlockSpec((_R, _NC, B, _D), lambda i: (0, 0, i, 0)),
            row_spec,
            pl.BlockSpec((1, _D), full),
            pl.BlockSpec((1, _D), full),
            pl.BlockSpec((1, _D), full),
            pl.BlockSpec((1, _D), full),
        ],
        out_specs=[row_spec, pl.BlockSpec((B, _R), lambda i: (i, 0))],
        out_shape=[jax.ShapeDtypeStruct((_N, _D), jnp.float32),
                   jax.ShapeDtypeStruct((_N, _R), jnp.float32)],
    )(part, sl, av2, bias2, gf, bf)


def _tgt_body(final_hbm, idx_hbm, out_hbm, idxv, rowsv, sem):
    wid = lax.axis_index("s") * _NC + lax.axis_index("c")
    base = wid * 32
    pltpu.sync_copy(idx_hbm.at[pl.ds(base, 32)], idxv)
    pltpu.async_copy(final_hbm.at[idxv], rowsv, sem).wait()
    pltpu.sync_copy(rowsv, out_hbm.at[pl.ds(base, 32)])


def _tgt_call(final_full, tgt):
    mesh = plsc.VectorSubcoreMesh(core_axis_name="c", subcore_axis_name="s")
    f = pl.kernel(
        _tgt_body,
        out_type=jax.ShapeDtypeStruct((1024, _D), jnp.float32),
        mesh=mesh,
        scratch_types=[
            pltpu.VMEM((32,), jnp.int32),
            pltpu.VMEM((32, _D), jnp.float32),
            pltpu.SemaphoreType.DMA,
        ],
    )
    return f(final_full, tgt)


def kernel(target_node_indices, emb, W_pre, b_pre, ln1_g, ln1_b, Wr, attn_vec,
           Ws, bs, bias, lnf_g, lnf_b, rows, cols, vals):
    bpre2 = b_pre.reshape(1, _D)
    g12 = ln1_g.reshape(1, _D)
    b12 = ln1_b.reshape(1, _D)
    bs2 = bs.reshape(1, _D)
    bias2 = bias.reshape(1, _D)
    gf2 = lnf_g.reshape(1, _D)
    bf2 = lnf_b.reshape(1, _D)
    av2 = attn_vec.reshape(1, _D)

    xr0, xr1, xr2, sl = _pre_call(emb, W_pre, bpre2, g12, b12, Wr, Ws, bs2)

    pad = _EPAD - _E
    rows_p = jnp.pad(rows, ((0, 0), (0, pad)))
    cols_p = jnp.pad(cols, ((0, 0), (0, pad)))
    vals_p = jnp.pad(vals, ((0, 0), (0, pad)))

    part = _spmm_call(xr0, xr1, xr2, rows_p, cols_p, vals_p)

    final_full, attn = _post_call(part, sl, av2, bias2, gf2, bf2)
    final = _tgt_call(final_full, target_node_indices)
    return final, attn, emb


# speedup vs baseline: 2.9277x; 1.0064x over previous
import functools

import jax
import jax.numpy as jnp
import numpy as np
from jax import lax
from jax.experimental import pallas as pl
from jax.experimental.pallas import tpu as pltpu
from jax.experimental.pallas import tpu_sc as plsc

_N = 10000
_D = 128
_R = 3
_E = 160000

_NC = 2
_NT = 16
_K = 160
_EPT = 5120
_EPAD = _EPT * _NC * _NT
_NCHUNK = _EPT // _K
_RPT = 640
_SQRT1_2 = np.float32(0.7071067811865476)


def _gelu(h):
    return 0.5 * h * (1.0 + lax.erf(h * _SQRT1_2))


def _layernorm_in(h, g, b):
    mu = jnp.mean(h, axis=1, keepdims=True)
    var = jnp.mean((h - mu) ** 2, axis=1, keepdims=True)
    return (h - mu) * lax.rsqrt(var + np.float32(1e-5)) * g + b


def _pre_body(emb_ref, wpre_ref, bpre_ref, g1_ref, b1_ref, wr_ref, ws_ref,
              bs_ref, xr0_ref, xr1_ref, xr2_ref, sl_ref):
    e = emb_ref[...]
    h = lax.dot_general(e, wpre_ref[...], (((1,), (1,)), ((), ())),
                        preferred_element_type=jnp.float32) + bpre_ref[...]
    h = _layernorm_in(h, g1_ref[...], b1_ref[...])
    x = _gelu(h)
    wr = wr_ref[...]
    xr0_ref[...] = lax.dot_general(x, wr[0], (((1,), (1,)), ((), ())),
                                   preferred_element_type=jnp.float32)
    xr1_ref[...] = lax.dot_general(x, wr[1], (((1,), (1,)), ((), ())),
                                   preferred_element_type=jnp.float32)
    xr2_ref[...] = lax.dot_general(x, wr[2], (((1,), (1,)), ((), ())),
                                   preferred_element_type=jnp.float32)
    sl_ref[...] = lax.dot_general(x, ws_ref[...], (((1,), (1,)), ((), ())),
                                  preferred_element_type=jnp.float32) + bs_ref[...]


def _pre_call(emb, wpre, bpre, g1, b1, wr, ws, bs2):
    B = 1000
    n_blk = _N // B
    full = lambda i: (0, 0)
    row_spec = pl.BlockSpec((B, _D), lambda i: (i, 0))
    return pl.pallas_call(
        _pre_body,
        grid=(n_blk,),
        in_specs=[
            row_spec,
            pl.BlockSpec((_D, _D), full),
            pl.BlockSpec((1, _D), full),
            pl.BlockSpec((1, _D), full),
            pl.BlockSpec((1, _D), full),
            pl.BlockSpec((_R, _D, _D), lambda i: (0, 0, 0)),
            pl.BlockSpec((_D, _D), full),
            pl.BlockSpec((1, _D), full),
        ],
        out_specs=[row_spec, row_spec, row_spec, row_spec],
        out_shape=[jax.ShapeDtypeStruct((_N, _D), jnp.float32)] * 4,
    )(emb, wpre, bpre, g1, b1, wr, ws, bs2)


def _scale_chunk(buf, valbuf):
    @pl.loop(0, _K // 16)
    def _grp(g):
        v16 = valbuf[pl.ds(g * 16, 16)]
        for j in range(16):
            vj = v16[j]
            k = g * 16 + j
            for t in range(_D // 16):
                buf[k, pl.ds(t * 16, 16)] = buf[k, pl.ds(t * 16, 16)] * vj


def _spmm_body(xr0_hbm, xr1_hbm, xr2_hbm,
               rows0, rows1, rows2, cols0, cols1, cols2, vals0, vals1, vals2,
               out_hbm, bufa, bufb, cola, colb, rowa, rowb, vala, valb,
               acc, gsema, gsemb):
    c = lax.axis_index("c")
    s = lax.axis_index("s")
    xr_list = [xr0_hbm, xr1_hbm, xr2_hbm]
    rows_list = [rows0, rows1, rows2]
    cols_list = [cols0, cols1, cols2]
    vals_list = [vals0, vals1, vals2]
    row0 = s * 624

    for r in range(_R):
        @pl.loop(0, _K)
        def _zero(k):
            for t in range(_D // 16):
                bufa[k, pl.ds(t * 16, 16)] = jnp.zeros((16,), jnp.float32)

        pltpu.sync_copy(bufa.at[pl.ds(0, 160), :], acc.at[pl.ds(row0, 160), :])
        pltpu.sync_copy(bufa.at[pl.ds(0, 160), :], acc.at[pl.ds(row0 + 160, 160), :])
        pltpu.sync_copy(bufa.at[pl.ds(0, 160), :], acc.at[pl.ds(row0 + 320, 160), :])
        pltpu.sync_copy(bufa.at[pl.ds(0, 144), :], acc.at[pl.ds(row0 + 480, 144), :])

        @pl.when(s == _NT - 1)
        def _zero_tail():
            pltpu.sync_copy(bufa.at[pl.ds(0, 16), :], acc.at[pl.ds(9984, 16), :])

        plsc.subcore_barrier()

        base = c * (_NT * _EPT) + s * _EPT
        xr = xr_list[r]
        rows_hbm = rows_list[r]
        cols_hbm = cols_list[r]
        vals_hbm = vals_list[r]

        def _fetch(i, colq, rowq, valq, bufq, semq):
            off = base + i * _K
            pltpu.sync_copy(cols_hbm.at[pl.ds(off, _K)], colq)
            pltpu.sync_copy(rows_hbm.at[pl.ds(off, _K)], rowq)
            pltpu.sync_copy(vals_hbm.at[pl.ds(off, _K)], valq)
            return pltpu.async_copy(xr.at[colq], bufq, semq)

        _fetch(0, cola, rowa, vala, bufa, gsema)

        @pl.loop(0, _NCHUNK // 2)
        def _pair(i):
            with jax.named_scope("fetchB"):
                hb = _fetch(2 * i + 1, colb, rowb, valb, bufb, gsemb)
            with jax.named_scope("gwaitA"):
                pltpu.make_async_copy(xr.at[cola], bufa, gsema).wait()
            with jax.named_scope("scaleA"):
                _scale_chunk(bufa, vala)
            with jax.named_scope("scatA"):
                pltpu.sync_copy(bufa, acc.at[rowa], add=True)

            @pl.when(i < _NCHUNK // 2 - 1)
            def _next_a():
                with jax.named_scope("fetchA"):
                    _fetch(2 * i + 2, cola, rowa, vala, bufa, gsema)

            with jax.named_scope("gwaitB"):
                hb.wait()
            with jax.named_scope("scaleB"):
                _scale_chunk(bufb, valb)
            with jax.named_scope("scatB"):
                pltpu.sync_copy(bufb, acc.at[rowb], add=True)

        plsc.subcore_barrier()
        pltpu.sync_copy(acc.at[pl.ds(row0, 624), :],
                        out_hbm.at[r, c, pl.ds(row0, 624), :])

        @pl.when(s == _NT - 1)
        def _wb_tail():
            pltpu.sync_copy(acc.at[pl.ds(9984, 16), :],
                            out_hbm.at[r, c, pl.ds(9984, 16), :])

        plsc.subcore_barrier()


def _spmm_call(xr0, xr1, xr2, rows_p, cols_p, vals_p):
    mesh = plsc.VectorSubcoreMesh(core_axis_name="c", subcore_axis_name="s")
    f = pl.kernel(
        _spmm_body,
        out_type=jax.ShapeDtypeStruct((_R, _NC, _N, _D), jnp.float32),
        mesh=mesh,
        scratch_types=[
            pltpu.VMEM((_K, _D), jnp.float32),
            pltpu.VMEM((_K, _D), jnp.float32),
            pltpu.VMEM((_K,), jnp.int32),
            pltpu.VMEM((_K,), jnp.int32),
            pltpu.VMEM((_K,), jnp.int32),
            pltpu.VMEM((_K,), jnp.int32),
            pltpu.VMEM((_K,), jnp.float32),
            pltpu.VMEM((_K,), jnp.float32),
            pltpu.VMEM_SHARED((_N, _D), jnp.float32),
            pltpu.SemaphoreType.DMA,
            pltpu.SemaphoreType.DMA,
        ],
    )
    return f(xr0, xr1, xr2, rows_p[0], rows_p[1], rows_p[2],
             cols_p[0], cols_p[1], cols_p[2], vals_p[0], vals_p[1], vals_p[2])


def _post_body(part_ref, sl_ref, av_ref, bias_ref, gf_ref, bf_ref,
               final_ref, attn_ref):
    m0 = part_ref[0, 0] + part_ref[0, 1]
    m1 = part_ref[1, 0] + part_ref[1, 1]
    m2 = part_ref[2, 0] + part_ref[2, 1]
    av = av_ref[...]
    s0 = jnp.sum(m0 * av, axis=1, keepdims=True)
    s1 = jnp.sum(m1 * av, axis=1, keepdims=True)
    s2 = jnp.sum(m2 * av, axis=1, keepdims=True)
    k0 = jnp.any(m0 != 0, axis=1, keepdims=True)
    k1 = jnp.any(m1 != 0, axis=1, keepdims=True)
    k2 = jnp.any(m2 != 0, axis=1, keepdims=True)
    neg = np.float32(-1e30)
    s0 = jnp.where(k0, s0, neg)
    s1 = jnp.where(k1, s1, neg)
    s2 = jnp.where(k2, s2, neg)
    mx = jnp.maximum(jnp.maximum(s0, s1), s2)
    e0 = jnp.where(k0, jnp.exp(s0 - mx), 0.0)
    e1 = jnp.where(k1, jnp.exp(s1 - mx), 0.0)
    e2 = jnp.where(k2, jnp.exp(s2 - mx), 0.0)
    den = e0 + e1 + e2
    inv = jnp.where(den > 0, 1.0 / den, 0.0)
    a0 = e0 * inv
    a1 = e1 * inv
    a2 = e2 * inv
    h = m0 * a0 + m1 * a1 + m2 * a2 + bias_ref[...] + sl_ref[...]
    out = _gelu(h)
    final_ref[...] = _layernorm_in(out, gf_ref[...], bf_ref[...])
    attn_ref[...] = jnp.concatenate([a0, a1, a2], axis=1)


def _post_call(part, sl, av2, bias2, gf, bf):
    B = 1000
    n_blk = _N // B
    full = lambda i: (0, 0)
    row_spec = pl.BlockSpec((B, _D), lambda i: (i, 0))
    return pl.pallas_call(
        _post_body,
        grid=(n_blk,),
        in_specs=[
            pl.BlockSpec((_R, _NC, B, _D), lambda i: (0, 0, i, 0)),
            row_spec,
            pl.BlockSpec((1, _D), full),
            pl.BlockSpec((1, _D), full),
            pl.BlockSpec((1, _D), full),
            pl.BlockSpec((1, _D), full),
        ],
        out_specs=[row_spec, pl.BlockSpec((B, _R), lambda i: (i, 0))],
        out_shape=[jax.ShapeDtypeStruct((_N, _D), jnp.float32),
                   jax.ShapeDtypeStruct((_N, _R), jnp.float32)],
    )(part, sl, av2, bias2, gf, bf)


def _tgt_body(final_hbm, idx_hbm, out_hbm, idxv, rowsv, sem):
    wid = lax.axis_index("s") * _NC + lax.axis_index("c")
    base = wid * 32
    pltpu.sync_copy(idx_hbm.at[pl.ds(base, 32)], idxv)
    pltpu.async_copy(final_hbm.at[idxv], rowsv, sem).wait()
    pltpu.sync_copy(rowsv, out_hbm.at[pl.ds(base, 32)])


def _tgt_call(final_full, tgt):
    mesh = plsc.VectorSubcoreMesh(core_axis_name="c", subcore_axis_name="s")
    f = pl.kernel(
        _tgt_body,
        out_type=jax.ShapeDtypeStruct((1024, _D), jnp.float32),
        mesh=mesh,
        scratch_types=[
            pltpu.VMEM((32,), jnp.int32),
            pltpu.VMEM((32, _D), jnp.float32),
            pltpu.SemaphoreType.DMA,
        ],
    )
    return f(final_full, tgt)


def kernel(target_node_indices, emb, W_pre, b_pre, ln1_g, ln1_b, Wr, attn_vec,
           Ws, bs, bias, lnf_g, lnf_b, rows, cols, vals):
    bpre2 = b_pre.reshape(1, _D)
    g12 = ln1_g.reshape(1, _D)
    b12 = ln1_b.reshape(1, _D)
    bs2 = bs.reshape(1, _D)
    bias2 = bias.reshape(1, _D)
    gf2 = lnf_g.reshape(1, _D)
    bf2 = lnf_b.reshape(1, _D)
    av2 = attn_vec.reshape(1, _D)

    xr0, xr1, xr2, sl = _pre_call(emb, W_pre, bpre2, g12, b12, Wr, Ws, bs2)

    pad = _EPAD - _E
    rows_p = jnp.pad(rows, ((0, 0), (0, pad)))
    cols_p = jnp.pad(cols, ((0, 0), (0, pad)))
    vals_p = jnp.pad(vals, ((0, 0), (0, pad)))

    part = _spmm_call(xr0, xr1, xr2, rows_p, cols_p, vals_p)

    final_full, attn = _post_call(part, sl, av2, bias2, gf2, bf2)
    final = _tgt_call(final_full, target_node_indices)
    return final, attn, emb


# speedup vs baseline: 5.4778x; 1.8710x over previous
import functools

import jax
import jax.numpy as jnp
import numpy as np
from jax import lax
from jax.experimental import pallas as pl
from jax.experimental.pallas import tpu as pltpu
from jax.experimental.pallas import tpu_sc as plsc

_N = 10000
_D = 128
_R = 3
_E = 160000

_NC = 2
_NT = 16
_K = 160
_EPT = 5120
_EPAD = _EPT * _NC * _NT
_NCHUNK = _EPT // _K
_RPT = 640
_SQRT1_2 = np.float32(0.7071067811865476)


def _gelu(h):
    return 0.5 * h * (1.0 + lax.erf(h * _SQRT1_2))


def _layernorm_in(h, g, b):
    mu = jnp.mean(h, axis=1, keepdims=True)
    var = jnp.mean((h - mu) ** 2, axis=1, keepdims=True)
    return (h - mu) * lax.rsqrt(var + np.float32(1e-5)) * g + b


def _pre_body(emb_ref, wpre_ref, bpre_ref, g1_ref, b1_ref, wr_ref, ws_ref,
              bs_ref, xr0_ref, xr1_ref, xr2_ref, sl_ref):
    e = emb_ref[...]
    h = lax.dot_general(e, wpre_ref[...], (((1,), (1,)), ((), ())),
                        preferred_element_type=jnp.float32) + bpre_ref[...]
    h = _layernorm_in(h, g1_ref[...], b1_ref[...])
    x = _gelu(h)
    wr = wr_ref[...]
    xr0_ref[...] = lax.dot_general(x, wr[0], (((1,), (1,)), ((), ())),
                                   preferred_element_type=jnp.float32)
    xr1_ref[...] = lax.dot_general(x, wr[1], (((1,), (1,)), ((), ())),
                                   preferred_element_type=jnp.float32)
    xr2_ref[...] = lax.dot_general(x, wr[2], (((1,), (1,)), ((), ())),
                                   preferred_element_type=jnp.float32)
    sl_ref[...] = lax.dot_general(x, ws_ref[...], (((1,), (1,)), ((), ())),
                                  preferred_element_type=jnp.float32) + bs_ref[...]


def _pre_call(emb, wpre, bpre, g1, b1, wr, ws, bs2):
    B = 1000
    n_blk = _N // B
    full = lambda i: (0, 0)
    row_spec = pl.BlockSpec((B, _D), lambda i: (i, 0))
    return pl.pallas_call(
        _pre_body,
        grid=(n_blk,),
        in_specs=[
            row_spec,
            pl.BlockSpec((_D, _D), full),
            pl.BlockSpec((1, _D), full),
            pl.BlockSpec((1, _D), full),
            pl.BlockSpec((1, _D), full),
            pl.BlockSpec((_R, _D, _D), lambda i: (0, 0, 0)),
            pl.BlockSpec((_D, _D), full),
            pl.BlockSpec((1, _D), full),
        ],
        out_specs=[row_spec, row_spec, row_spec, row_spec],
        out_shape=[jax.ShapeDtypeStruct((_N, _D), jnp.float32)] * 4,
    )(emb, wpre, bpre, g1, b1, wr, ws, bs2)


def _scale_chunk(buf, valbuf):
    @pl.loop(0, _K // 16)
    def _grp(g):
        v16 = valbuf[pl.ds(g * 16, 16)]
        for j in range(16):
            vj = v16[j]
            k = g * 16 + j
            for t in range(_D // 16):
                buf[k, pl.ds(t * 16, 16)] = buf[k, pl.ds(t * 16, 16)] * vj


def _spmm_body(xr0_hbm, xr1_hbm, xr2_hbm,
               rows0, rows1, rows2, cols0, cols1, cols2, vals0, vals1, vals2,
               out_hbm, bufa, bufb, cola, colb, rowa, rowb, vala, valb,
               acc, gsema, gsemb):
    c = lax.axis_index("c")
    s = lax.axis_index("s")
    xr_list = [xr0_hbm, xr1_hbm, xr2_hbm]
    rows_list = [rows0, rows1, rows2]
    cols_list = [cols0, cols1, cols2]
    vals_list = [vals0, vals1, vals2]
    row0 = s * 624

    for r in range(_R):
        @pl.loop(0, _K)
        def _zero(k):
            for t in range(_D // 16):
                bufa[k, pl.ds(t * 16, 16)] = jnp.zeros((16,), jnp.float32)

        pltpu.sync_copy(bufa.at[pl.ds(0, 160), :], acc.at[pl.ds(row0, 160), :])
        pltpu.sync_copy(bufa.at[pl.ds(0, 160), :], acc.at[pl.ds(row0 + 160, 160), :])
        pltpu.sync_copy(bufa.at[pl.ds(0, 160), :], acc.at[pl.ds(row0 + 320, 160), :])
        pltpu.sync_copy(bufa.at[pl.ds(0, 144), :], acc.at[pl.ds(row0 + 480, 144), :])

        @pl.when(s == _NT - 1)
        def _zero_tail():
            pltpu.sync_copy(bufa.at[pl.ds(0, 16), :], acc.at[pl.ds(9984, 16), :])

        plsc.subcore_barrier()

        base = c * (_NT * _EPT) + s * _EPT
        xr = xr_list[r]
        rows_hbm = rows_list[r]
        cols_hbm = cols_list[r]
        vals_hbm = vals_list[r]

        def _fetch(i, colq, rowq, valq, bufq, semq):
            off = base + i * _K
            pltpu.sync_copy(cols_hbm.at[pl.ds(off, _K)], colq)
            pltpu.sync_copy(rows_hbm.at[pl.ds(off, _K)], rowq)
            pltpu.sync_copy(vals_hbm.at[pl.ds(off, _K)], valq)
            return pltpu.async_copy(xr.at[colq], bufq, semq)

        _fetch(0, cola, rowa, vala, bufa, gsema)

        @pl.loop(0, _NCHUNK // 2)
        def _pair(i):
            with jax.named_scope("fetchB"):
                hb = _fetch(2 * i + 1, colb, rowb, valb, bufb, gsemb)
            with jax.named_scope("gwaitA"):
                pltpu.make_async_copy(xr.at[cola], bufa, gsema).wait()
            with jax.named_scope("scaleA"):
                _scale_chunk(bufa, vala)
            with jax.named_scope("scatA"):
                pltpu.sync_copy(bufa, acc.at[rowa], add=True)

            @pl.when(i < _NCHUNK // 2 - 1)
            def _next_a():
                with jax.named_scope("fetchA"):
                    _fetch(2 * i + 2, cola, rowa, vala, bufa, gsema)

            with jax.named_scope("gwaitB"):
                hb.wait()
            with jax.named_scope("scaleB"):
                _scale_chunk(bufb, valb)
            with jax.named_scope("scatB"):
                pltpu.sync_copy(bufb, acc.at[rowb], add=True)

        plsc.subcore_barrier()
        pltpu.sync_copy(acc.at[pl.ds(row0, 624), :],
                        out_hbm.at[r, c, pl.ds(row0, 624), :])

        @pl.when(s == _NT - 1)
        def _wb_tail():
            pltpu.sync_copy(acc.at[pl.ds(9984, 16), :],
                            out_hbm.at[r, c, pl.ds(9984, 16), :])

        plsc.subcore_barrier()


def _spmm_call(xr0, xr1, xr2, rows_p, cols_p, vals_p):
    mesh = plsc.VectorSubcoreMesh(core_axis_name="c", subcore_axis_name="s")
    f = pl.kernel(
        _spmm_body,
        out_type=jax.ShapeDtypeStruct((_R, _NC, _N, _D), jnp.float32),
        mesh=mesh,
        scratch_types=[
            pltpu.VMEM((_K, _D), jnp.float32),
            pltpu.VMEM((_K, _D), jnp.float32),
            pltpu.VMEM((_K,), jnp.int32),
            pltpu.VMEM((_K,), jnp.int32),
            pltpu.VMEM((_K,), jnp.int32),
            pltpu.VMEM((_K,), jnp.int32),
            pltpu.VMEM((_K,), jnp.float32),
            pltpu.VMEM((_K,), jnp.float32),
            pltpu.VMEM_SHARED((_N, _D), jnp.float32),
            pltpu.SemaphoreType.DMA,
            pltpu.SemaphoreType.DMA,
        ],
    )
    return f(xr0, xr1, xr2, rows_p[0], rows_p[1], rows_p[2],
             cols_p[0], cols_p[1], cols_p[2], vals_p[0], vals_p[1], vals_p[2])


def _post_body(part_ref, sl_ref, av_ref, bias_ref, gf_ref, bf_ref,
               final_ref, attn_ref):
    m0 = part_ref[0, 0] + part_ref[0, 1]
    m1 = part_ref[1, 0] + part_ref[1, 1]
    m2 = part_ref[2, 0] + part_ref[2, 1]
    av = av_ref[...]
    s0 = jnp.sum(m0 * av, axis=1, keepdims=True)
    s1 = jnp.sum(m1 * av, axis=1, keepdims=True)
    s2 = jnp.sum(m2 * av, axis=1, keepdims=True)
    k0 = jnp.any(m0 != 0, axis=1, keepdims=True)
    k1 = jnp.any(m1 != 0, axis=1, keepdims=True)
    k2 = jnp.any(m2 != 0, axis=1, keepdims=True)
    neg = np.float32(-1e30)
    s0 = jnp.where(k0, s0, neg)
    s1 = jnp.where(k1, s1, neg)
    s2 = jnp.where(k2, s2, neg)
    mx = jnp.maximum(jnp.maximum(s0, s1), s2)
    e0 = jnp.where(k0, jnp.exp(s0 - mx), 0.0)
    e1 = jnp.where(k1, jnp.exp(s1 - mx), 0.0)
    e2 = jnp.where(k2, jnp.exp(s2 - mx), 0.0)
    den = e0 + e1 + e2
    inv = jnp.where(den > 0, 1.0 / den, 0.0)
    a0 = e0 * inv
    a1 = e1 * inv
    a2 = e2 * inv
    h = m0 * a0 + m1 * a1 + m2 * a2 + bias_ref[...] + sl_ref[...]
    out = _gelu(h)
    final_ref[...] = _layernorm_in(out, gf_ref[...], bf_ref[...])
    attn_ref[...] = jnp.concatenate([a0, a1, a2], axis=1)


def _post_call(part, sl, av2, bias2, gf, bf):
    B = 1000
    n_blk = _N // B
    full = lambda i: (0, 0)
    row_spec = pl.BlockSpec((B, _D), lambda i: (i, 0))
    return pl.pallas_call(
        _post_body,
        grid=(n_blk,),
        in_specs=[
            pl.BlockSpec((_R, _NC, B, _D), lambda i: (0, 0, i, 0)),
            row_spec,
            pl.BlockSpec((1, _D), full),
            pl.BlockSpec((1, _D), full),
            pl.BlockSpec((1, _D), full),
            pl.BlockSpec((1, _D), full),
        ],
        out_specs=[row_spec, pl.BlockSpec((B, _R), lambda i: (i, 0))],
        out_shape=[jax.ShapeDtypeStruct((_N, _D), jnp.float32),
                   jax.ShapeDtypeStruct((_N, _R), jnp.float32)],
    )(part, sl, av2, bias2, gf, bf)


def _tgt_body(final_hbm, idx_hbm, out_hbm, idxv, rowsv, sem):
    wid = lax.axis_index("s") * _NC + lax.axis_index("c")
    base = wid * 32
    pltpu.sync_copy(idx_hbm.at[pl.ds(base, 32)], idxv)
    pltpu.async_copy(final_hbm.at[idxv], rowsv, sem).wait()
    pltpu.sync_copy(rowsv, out_hbm.at[pl.ds(base, 32)])


def _tgt_call(final_full, tgt):
    mesh = plsc.VectorSubcoreMesh(core_axis_name="c", subcore_axis_name="s")
    f = pl.kernel(
        _tgt_body,
        out_type=jax.ShapeDtypeStruct((1024, _D), jnp.float32),
        mesh=mesh,
        scratch_types=[
            pltpu.VMEM((32,), jnp.int32),
            pltpu.VMEM((32, _D), jnp.float32),
            pltpu.SemaphoreType.DMA,
        ],
    )
    return f(final_full, tgt)


def kernel(target_node_indices, emb, W_pre, b_pre, ln1_g, ln1_b, Wr, attn_vec,
           Ws, bs, bias, lnf_g, lnf_b, rows, cols, vals):
    bpre2 = b_pre.reshape(1, _D)
    g12 = ln1_g.reshape(1, _D)
    b12 = ln1_b.reshape(1, _D)
    bs2 = bs.reshape(1, _D)
    bias2 = bias.reshape(1, _D)
    gf2 = lnf_g.reshape(1, _D)
    bf2 = lnf_b.reshape(1, _D)
    av2 = attn_vec.reshape(1, _D)

    xr0, xr1, xr2, sl = _pre_call(emb, W_pre, bpre2, g12, b12, Wr, Ws, bs2)

    nw = _NC * _NT
    real = _E // nw
    padw = _EPT - real
    r3 = rows.reshape(_R, nw, real)
    c3 = cols.reshape(_R, nw, real)
    v3 = vals.reshape(_R, nw, real)
    pidx = (jnp.arange(nw, dtype=jnp.int32)[None, :, None] * padw
            + jnp.arange(padw, dtype=jnp.int32)[None, None, :])
    pidx = jnp.broadcast_to(pidx, (_R, nw, padw))
    rows_p = jnp.concatenate([r3, pidx], axis=2).reshape(_R, _EPAD)
    cols_p = jnp.concatenate([c3, pidx], axis=2).reshape(_R, _EPAD)
    vals_p = jnp.concatenate(
        [v3, jnp.zeros((_R, nw, padw), jnp.float32)], axis=2).reshape(_R, _EPAD)

    part = _spmm_call(xr0, xr1, xr2, rows_p, cols_p, vals_p)

    final_full, attn = _post_call(part, sl, av2, bias2, gf2, bf2)
    final = _tgt_call(final_full, target_node_indices)
    return final, attn, emb


# speedup vs baseline: 7.0059x; 1.2790x over previous
import functools

import jax
import jax.numpy as jnp
import numpy as np
from jax import lax
from jax.experimental import pallas as pl
from jax.experimental.pallas import tpu as pltpu
from jax.experimental.pallas import tpu_sc as plsc

_N = 10000
_D = 128
_R = 3
_E = 160000

_NC = 2
_NT = 16
_K = 112
_CPT = 45
_EPT = _K * _CPT
_EPAD = _EPT * _NC * _NT
_SQRT1_2 = np.float32(0.7071067811865476)


def _gelu(h):
    return 0.5 * h * (1.0 + lax.erf(h * _SQRT1_2))


def _layernorm_in(h, g, b):
    mu = jnp.mean(h, axis=1, keepdims=True)
    var = jnp.mean((h - mu) ** 2, axis=1, keepdims=True)
    return (h - mu) * lax.rsqrt(var + np.float32(1e-5)) * g + b


def _pre_body(emb_ref, wpre_ref, bpre_ref, g1_ref, b1_ref, wr_ref, ws_ref,
              bs_ref, xr0_ref, xr1_ref, xr2_ref, sl_ref):
    e = emb_ref[...]
    h = lax.dot_general(e, wpre_ref[...], (((1,), (1,)), ((), ())),
                        preferred_element_type=jnp.float32) + bpre_ref[...]
    h = _layernorm_in(h, g1_ref[...], b1_ref[...])
    x = _gelu(h)
    wr = wr_ref[...]
    xr0_ref[...] = lax.dot_general(x, wr[0], (((1,), (1,)), ((), ())),
                                   preferred_element_type=jnp.float32)
    xr1_ref[...] = lax.dot_general(x, wr[1], (((1,), (1,)), ((), ())),
                                   preferred_element_type=jnp.float32)
    xr2_ref[...] = lax.dot_general(x, wr[2], (((1,), (1,)), ((), ())),
                                   preferred_element_type=jnp.float32)
    sl_ref[...] = lax.dot_general(x, ws_ref[...], (((1,), (1,)), ((), ())),
                                  preferred_element_type=jnp.float32) + bs_ref[...]


def _pre_call(emb, wpre, bpre, g1, b1, wr, ws, bs2):
    B = 1000
    n_blk = _N // B
    full = lambda i: (0, 0)
    row_spec = pl.BlockSpec((B, _D), lambda i: (i, 0))
    return pl.pallas_call(
        _pre_body,
        grid=(n_blk,),
        in_specs=[
            row_spec,
            pl.BlockSpec((_D, _D), full),
            pl.BlockSpec((1, _D), full),
            pl.BlockSpec((1, _D), full),
            pl.BlockSpec((1, _D), full),
            pl.BlockSpec((_R, _D, _D), lambda i: (0, 0, 0)),
            pl.BlockSpec((_D, _D), full),
            pl.BlockSpec((1, _D), full),
        ],
        out_specs=[row_spec, row_spec, row_spec, row_spec],
        out_shape=[jax.ShapeDtypeStruct((_N, _D), jnp.float32)] * 4,
    )(emb, wpre, bpre, g1, b1, wr, ws, bs2)


def _scale_chunk(buf, valbuf):
    @pl.loop(0, _K // 16)
    def _grp(g):
        v16 = valbuf[pl.ds(g * 16, 16)]
        for j in range(16):
            vj = v16[j]
            k = g * 16 + j
            for t in range(_D // 16):
                buf[k, pl.ds(t * 16, 16)] = buf[k, pl.ds(t * 16, 16)] * vj


_NTRIP = _CPT // 3


def _spmm_body(xr0_hbm, xr1_hbm, xr2_hbm,
               rows0, rows1, rows2, cols0, cols1, cols2, vals0, vals1, vals2,
               out_hbm,
               buf0, buf1, buf2, col0, col1, col2, row0b, row1b, row2b,
               val0, val1, val2, acc,
               gsem0, gsem1, gsem2, esem0, esem1, esem2, ssem0, ssem1, ssem2):
    c = lax.axis_index("c")
    s = lax.axis_index("s")
    xr_list = [xr0_hbm, xr1_hbm, xr2_hbm]
    rows_list = [rows0, rows1, rows2]
    cols_list = [cols0, cols1, cols2]
    vals_list = [vals0, vals1, vals2]
    bufs = [buf0, buf1, buf2]
    colb = [col0, col1, col2]
    rowb = [row0b, row1b, row2b]
    valb = [val0, val1, val2]
    gsems = [gsem0, gsem1, gsem2]
    esems = [esem0, esem1, esem2]
    ssems = [ssem0, ssem1, ssem2]
    row0 = s * 624

    for r in range(_R):
        @pl.loop(0, _K)
        def _zero(k):
            for t in range(_D // 16):
                buf0[k, pl.ds(t * 16, 16)] = jnp.zeros((16,), jnp.float32)

        for z in range(5):
            pltpu.sync_copy(buf0.at[pl.ds(0, _K), :],
                            acc.at[pl.ds(row0 + z * _K, _K), :])
        pltpu.sync_copy(buf0.at[pl.ds(0, 64), :], acc.at[pl.ds(row0 + 560, 64), :])

        @pl.when(s == _NT - 1)
        def _zero_tail():
            pltpu.sync_copy(buf0.at[pl.ds(0, 16), :], acc.at[pl.ds(9984, 16), :])

        plsc.subcore_barrier()

        base = c * (_NT * _EPT) + s * _EPT
        xr = xr_list[r]
        rows_hbm = rows_list[r]
        cols_hbm = cols_list[r]
        vals_hbm = vals_list[r]

        def _estart(i, b):
            off = base + i * _K
            pltpu.async_copy(cols_hbm.at[pl.ds(off, _K)], colb[b], esems[b])
            pltpu.async_copy(rows_hbm.at[pl.ds(off, _K)], rowb[b], esems[b])
            pltpu.async_copy(vals_hbm.at[pl.ds(off, _K)], valb[b], esems[b])

        def _ewait_gstart(b):
            pltpu.make_async_copy(cols_hbm.at[pl.ds(base, _K)], colb[b], esems[b]).wait()
            pltpu.make_async_copy(rows_hbm.at[pl.ds(base, _K)], rowb[b], esems[b]).wait()
            pltpu.make_async_copy(vals_hbm.at[pl.ds(base, _K)], valb[b], esems[b]).wait()
            pltpu.async_copy(xr.at[colb[b]], bufs[b], gsems[b])

        def _gwait(b):
            pltpu.make_async_copy(xr.at[colb[b]], bufs[b], gsems[b]).wait()

        def _sstart(b):
            pltpu.async_copy(bufs[b], acc.at[rowb[b]], ssems[b], add=True)

        def _swait(b):
            pltpu.make_async_copy(bufs[b], acc.at[rowb[b]], ssems[b]).wait()

        _estart(0, 0)
        _estart(1, 1)
        _ewait_gstart(0)

        @pl.loop(0, _NTRIP)
        def _triple(j):
            for t in range(3):
                b = t
                b1 = (t + 1) % 3
                bp = (t + 2) % 3
                if t < 2:
                    _ewait_gstart(b1)
                else:
                    @pl.when(j < _NTRIP - 1)
                    def _head():
                        _ewait_gstart(b1)
                _gwait(b)
                _scale_chunk(bufs[b], valb[b])
                _sstart(b)
                if t == 0:
                    @pl.when(j > 0)
                    def _t0w():
                        _swait(bp)
                    _estart(3 * j + 2, bp)
                else:
                    @pl.when(j < _NTRIP - 1)
                    def _tail():
                        _swait(bp)
                        _estart(3 * j + t + 2, bp)

        _swait(0)
        _swait(1)
        _swait(2)

        plsc.subcore_barrier()
        pltpu.sync_copy(acc.at[pl.ds(row0, 624), :],
                        out_hbm.at[r, c, pl.ds(row0, 624), :])

        @pl.when(s == _NT - 1)
        def _wb_tail():
            pltpu.sync_copy(acc.at[pl.ds(9984, 16), :],
                            out_hbm.at[r, c, pl.ds(9984, 16), :])

        plsc.subcore_barrier()


def _spmm_call(xr0, xr1, xr2, rows_p, cols_p, vals_p):
    mesh = plsc.VectorSubcoreMesh(core_axis_name="c", subcore_axis_name="s")
    f = pl.kernel(
        _spmm_body,
        out_type=jax.ShapeDtypeStruct((_R, _NC, _N, _D), jnp.float32),
        mesh=mesh,
        scratch_types=(
            [pltpu.VMEM((_K, _D), jnp.float32)] * 3
            + [pltpu.VMEM((_K,), jnp.int32)] * 6
            + [pltpu.VMEM((_K,), jnp.float32)] * 3
            + [pltpu.VMEM_SHARED((_N, _D), jnp.float32)]
            + [pltpu.SemaphoreType.DMA] * 9
        ),
    )
    return f(xr0, xr1, xr2, rows_p[0], rows_p[1], rows_p[2],
             cols_p[0], cols_p[1], cols_p[2], vals_p[0], vals_p[1], vals_p[2])


def _post_body(part_ref, sl_ref, av_ref, bias_ref, gf_ref, bf_ref,
               final_ref, attn_ref):
    m0 = part_ref[0, 0] + part_ref[0, 1]
    m1 = part_ref[1, 0] + part_ref[1, 1]
    m2 = part_ref[2, 0] + part_ref[2, 1]
    av = av_ref[...]
    s0 = jnp.sum(m0 * av, axis=1, keepdims=True)
    s1 = jnp.sum(m1 * av, axis=1, keepdims=True)
    s2 = jnp.sum(m2 * av, axis=1, keepdims=True)
    k0 = jnp.any(m0 != 0, axis=1, keepdims=True)
    k1 = jnp.any(m1 != 0, axis=1, keepdims=True)
    k2 = jnp.any(m2 != 0, axis=1, keepdims=True)
    neg = np.float32(-1e30)
    s0 = jnp.where(k0, s0, neg)
    s1 = jnp.where(k1, s1, neg)
    s2 = jnp.where(k2, s2, neg)
    mx = jnp.maximum(jnp.maximum(s0, s1), s2)
    e0 = jnp.where(k0, jnp.exp(s0 - mx), 0.0)
    e1 = jnp.where(k1, jnp.exp(s1 - mx), 0.0)
    e2 = jnp.where(k2, jnp.exp(s2 - mx), 0.0)
    den = e0 + e1 + e2
    inv = jnp.where(den > 0, 1.0 / den, 0.0)
    a0 = e0 * inv
    a1 = e1 * inv
    a2 = e2 * inv
    h = m0 * a0 + m1 * a1 + m2 * a2 + bias_ref[...] + sl_ref[...]
    out = _gelu(h)
    final_ref[...] = _layernorm_in(out, gf_ref[...], bf_ref[...])
    attn_ref[...] = jnp.concatenate([a0, a1, a2], axis=1)


def _post_call(part, sl, av2, bias2, gf, bf):
    B = 1000
    n_blk = _N // B
    full = lambda i: (0, 0)
    row_spec = pl.BlockSpec((B, _D), lambda i: (i, 0))
    return pl.pallas_call(
        _post_body,
        grid=(n_blk,),
        in_specs=[
            pl.BlockSpec((_R, _NC, B, _D), lambda i: (0, 0, i, 0)),
            row_spec,
            pl.BlockSpec((1, _D), full),
            pl.BlockSpec((1, _D), full),
            pl.BlockSpec((1, _D), full),
            pl.BlockSpec((1, _D), full),
        ],
        out_specs=[row_spec, pl.BlockSpec((B, _R), lambda i: (i, 0))],
        out_shape=[jax.ShapeDtypeStruct((_N, _D), jnp.float32),
                   jax.ShapeDtypeStruct((_N, _R), jnp.float32)],
    )(part, sl, av2, bias2, gf, bf)


def _tgt_body(final_hbm, idx_hbm, out_hbm, idxv, rowsv, sem):
    wid = lax.axis_index("s") * _NC + lax.axis_index("c")
    base = wid * 32
    pltpu.sync_copy(idx_hbm.at[pl.ds(base, 32)], idxv)
    pltpu.async_copy(final_hbm.at[idxv], rowsv, sem).wait()
    pltpu.sync_copy(rowsv, out_hbm.at[pl.ds(base, 32)])


def _tgt_call(final_full, tgt):
    mesh = plsc.VectorSubcoreMesh(core_axis_name="c", subcore_axis_name="s")
    f = pl.kernel(
        _tgt_body,
        out_type=jax.ShapeDtypeStruct((1024, _D), jnp.float32),
        mesh=mesh,
        scratch_types=[
            pltpu.VMEM((32,), jnp.int32),
            pltpu.VMEM((32, _D), jnp.float32),
            pltpu.SemaphoreType.DMA,
        ],
    )
    return f(final_full, tgt)


def kernel(target_node_indices, emb, W_pre, b_pre, ln1_g, ln1_b, Wr, attn_vec,
           Ws, bs, bias, lnf_g, lnf_b, rows, cols, vals):
    bpre2 = b_pre.reshape(1, _D)
    g12 = ln1_g.reshape(1, _D)
    b12 = ln1_b.reshape(1, _D)
    bs2 = bs.reshape(1, _D)
    bias2 = bias.reshape(1, _D)
    gf2 = lnf_g.reshape(1, _D)
    bf2 = lnf_b.reshape(1, _D)
    av2 = attn_vec.reshape(1, _D)

    xr0, xr1, xr2, sl = _pre_call(emb, W_pre, bpre2, g12, b12, Wr, Ws, bs2)

    nw = _NC * _NT
    real = _E // nw
    padw = _EPT - real
    r3 = rows.reshape(_R, nw, real)
    c3 = cols.reshape(_R, nw, real)
    v3 = vals.reshape(_R, nw, real)
    pidx = (jnp.arange(nw, dtype=jnp.int32)[None, :, None] * padw
            + jnp.arange(padw, dtype=jnp.int32)[None, None, :])
    pidx = jnp.broadcast_to(pidx, (_R, nw, padw))
    rows_p = jnp.concatenate([r3, pidx], axis=2).reshape(_R, _EPAD)
    cols_p = jnp.concatenate([c3, pidx], axis=2).reshape(_R, _EPAD)
    vals_p = jnp.concatenate(
        [v3, jnp.zeros((_R, nw, padw), jnp.float32)], axis=2).reshape(_R, _EPAD)

    part = _spmm_call(xr0, xr1, xr2, rows_p, cols_p, vals_p)

    final_full, attn = _post_call(part, sl, av2, bias2, gf2, bf2)
    final = _tgt_call(final_full, target_node_indices)
    return final, attn, emb


# speedup vs baseline: 7.6261x; 1.0885x over previous
import functools

import jax
import jax.numpy as jnp
import numpy as np
from jax import lax
from jax.experimental import pallas as pl
from jax.experimental.pallas import tpu as pltpu
from jax.experimental.pallas import tpu_sc as plsc

_N = 10000
_D = 128
_R = 3
_E = 160000

_NC = 2
_NT = 16
_K = 112
_CPT = 45
_EPT = _K * _CPT
_EPAD = _EPT * _NC * _NT
_SQRT1_2 = np.float32(0.7071067811865476)


def _gelu(h):
    return 0.5 * h * (1.0 + lax.erf(h * _SQRT1_2))


def _layernorm_in(h, g, b):
    mu = jnp.mean(h, axis=1, keepdims=True)
    var = jnp.mean((h - mu) ** 2, axis=1, keepdims=True)
    return (h - mu) * lax.rsqrt(var + np.float32(1e-5)) * g + b


def _pre_body(emb_ref, wpre_ref, bpre_ref, g1_ref, b1_ref, wcat_ref,
              bs_ref, xr0_ref, xr1_ref, xr2_ref, sl_ref):
    e = emb_ref[...]
    h = lax.dot_general(e, wpre_ref[...], (((1,), (1,)), ((), ())),
                        preferred_element_type=jnp.float32) + bpre_ref[...]
    h = _layernorm_in(h, g1_ref[...], b1_ref[...])
    x = _gelu(h)
    h2 = lax.dot_general(x, wcat_ref[...], (((1,), (1,)), ((), ())),
                         preferred_element_type=jnp.float32)
    xr0_ref[...] = h2[:, :_D]
    xr1_ref[...] = h2[:, _D:2 * _D]
    xr2_ref[...] = h2[:, 2 * _D:3 * _D]
    sl_ref[...] = h2[:, 3 * _D:] + bs_ref[...]


def _pre_call(emb, wpre, bpre, g1, b1, wcat, bs2):
    B = 1000
    n_blk = _N // B
    full = lambda i: (0, 0)
    row_spec = pl.BlockSpec((B, _D), lambda i: (i, 0))
    return pl.pallas_call(
        _pre_body,
        grid=(n_blk,),
        in_specs=[
            row_spec,
            pl.BlockSpec((_D, _D), full),
            pl.BlockSpec((1, _D), full),
            pl.BlockSpec((1, _D), full),
            pl.BlockSpec((1, _D), full),
            pl.BlockSpec((4 * _D, _D), full),
            pl.BlockSpec((1, _D), full),
        ],
        out_specs=[row_spec, row_spec, row_spec, row_spec],
        out_shape=[jax.ShapeDtypeStruct((_N, _D), jnp.float32)] * 4,
    )(emb, wpre, bpre, g1, b1, wcat, bs2)


def _scale_chunk(buf, valbuf):
    @pl.loop(0, _K // 16)
    def _grp(g):
        v16 = valbuf[pl.ds(g * 16, 16)]
        for j in range(16):
            vj = v16[j]
            k = g * 16 + j
            for t in range(_D // 16):
                buf[k, pl.ds(t * 16, 16)] = buf[k, pl.ds(t * 16, 16)] * vj


_NTRIP = _CPT // 3


def _spmm_body(xr0_hbm, xr1_hbm, xr2_hbm,
               rows_hbm, cols_hbm, vals_hbm,
               out_hbm,
               buf0, buf1, buf2, col0, col1, col2, row0b, row1b, row2b,
               val0, val1, val2, acc,
               gsem0, gsem1, gsem2, esem0, esem1, esem2, ssem0, ssem1, ssem2):
    c = lax.axis_index("c")
    s = lax.axis_index("s")
    xr_list = [xr0_hbm, xr1_hbm, xr2_hbm]
    bufs = [buf0, buf1, buf2]
    colb = [col0, col1, col2]
    rowb = [row0b, row1b, row2b]
    valb = [val0, val1, val2]
    gsems = [gsem0, gsem1, gsem2]
    esems = [esem0, esem1, esem2]
    ssems = [ssem0, ssem1, ssem2]
    row0 = s * 624

    for r in range(_R):
        @pl.loop(0, _K)
        def _zero(k):
            for t in range(_D // 16):
                buf0[k, pl.ds(t * 16, 16)] = jnp.zeros((16,), jnp.float32)

        for z in range(5):
            pltpu.sync_copy(buf0.at[pl.ds(0, _K), :],
                            acc.at[pl.ds(row0 + z * _K, _K), :])
        pltpu.sync_copy(buf0.at[pl.ds(0, 64), :], acc.at[pl.ds(row0 + 560, 64), :])

        @pl.when(s == _NT - 1)
        def _zero_tail():
            pltpu.sync_copy(buf0.at[pl.ds(0, 16), :], acc.at[pl.ds(9984, 16), :])

        plsc.subcore_barrier()

        base = r * _EPAD + (c * _NT + s) * _EPT
        xr = xr_list[r]

        def _estart(i, b):
            off = base + i * _K
            pltpu.async_copy(cols_hbm.at[pl.ds(off, _K)], colb[b], esems[b])
            pltpu.async_copy(rows_hbm.at[pl.ds(off, _K)], rowb[b], esems[b])
            pltpu.async_copy(vals_hbm.at[pl.ds(off, _K)], valb[b], esems[b])

        def _ewait_gstart(b):
            pltpu.make_async_copy(cols_hbm.at[pl.ds(base, _K)], colb[b], esems[b]).wait()
            pltpu.make_async_copy(rows_hbm.at[pl.ds(base, _K)], rowb[b], esems[b]).wait()
            pltpu.make_async_copy(vals_hbm.at[pl.ds(base, _K)], valb[b], esems[b]).wait()
            pltpu.async_copy(xr.at[colb[b]], bufs[b], gsems[b])

        def _gwait(b):
            pltpu.make_async_copy(xr.at[colb[b]], bufs[b], gsems[b]).wait()

        def _sstart(b):
            pltpu.async_copy(bufs[b], acc.at[rowb[b]], ssems[b], add=True)

        def _swait(b):
            pltpu.make_async_copy(bufs[b], acc.at[rowb[b]], ssems[b]).wait()

        _estart(0, 0)
        _estart(1, 1)
        _ewait_gstart(0)

        @pl.loop(0, _NTRIP)
        def _triple(j):
            for t in range(3):
                b = t
                b1 = (t + 1) % 3
                bp = (t + 2) % 3
                if t < 2:
                    _ewait_gstart(b1)
                else:
                    @pl.when(j < _NTRIP - 1)
                    def _head():
                        _ewait_gstart(b1)
                _gwait(b)
                _scale_chunk(bufs[b], valb[b])
                _sstart(b)
                if t == 0:
                    @pl.when(j > 0)
                    def _t0w():
                        _swait(bp)
                    _estart(3 * j + 2, bp)
                else:
                    @pl.when(j < _NTRIP - 1)
                    def _tail():
                        _swait(bp)
                        _estart(3 * j + t + 2, bp)

        _swait(0)
        _swait(1)
        _swait(2)

        plsc.subcore_barrier()
        pltpu.sync_copy(acc.at[pl.ds(row0, 624), :],
                        out_hbm.at[r, c, pl.ds(row0, 624), :])

        @pl.when(s == _NT - 1)
        def _wb_tail():
            pltpu.sync_copy(acc.at[pl.ds(9984, 16), :],
                            out_hbm.at[r, c, pl.ds(9984, 16), :])

        plsc.subcore_barrier()


def _spmm_call(xr0, xr1, xr2, rows_f, cols_f, vals_f):
    mesh = plsc.VectorSubcoreMesh(core_axis_name="c", subcore_axis_name="s",
                                  num_cores=_NC)
    f = pl.kernel(
        _spmm_body,
        out_type=jax.ShapeDtypeStruct((_R, _NC, _N, _D), jnp.float32),
        mesh=mesh,
        scratch_types=(
            [pltpu.VMEM((_K, _D), jnp.float32)] * 3
            + [pltpu.VMEM((_K,), jnp.int32)] * 6
            + [pltpu.VMEM((_K,), jnp.float32)] * 3
            + [pltpu.VMEM_SHARED((_N, _D), jnp.float32)]
            + [pltpu.SemaphoreType.DMA] * 9
        ),
    )
    return f(xr0, xr1, xr2, rows_f, cols_f, vals_f)


def _post_body(part_ref, sl_ref, av_ref, bias_ref, gf_ref, bf_ref,
               final_ref, attn_ref):
    m0 = part_ref[0, 0] + part_ref[0, 1]
    m1 = part_ref[1, 0] + part_ref[1, 1]
    m2 = part_ref[2, 0] + part_ref[2, 1]
    av = av_ref[...]
    s0 = jnp.sum(m0 * av, axis=1, keepdims=True)
    s1 = jnp.sum(m1 * av, axis=1, keepdims=True)
    s2 = jnp.sum(m2 * av, axis=1, keepdims=True)
    k0 = jnp.any(m0 != 0, axis=1, keepdims=True)
    k1 = jnp.any(m1 != 0, axis=1, keepdims=True)
    k2 = jnp.any(m2 != 0, axis=1, keepdims=True)
    neg = np.float32(-1e30)
    s0 = jnp.where(k0, s0, neg)
    s1 = jnp.where(k1, s1, neg)
    s2 = jnp.where(k2, s2, neg)
    mx = jnp.maximum(jnp.maximum(s0, s1), s2)
    e0 = jnp.where(k0, jnp.exp(s0 - mx), 0.0)
    e1 = jnp.where(k1, jnp.exp(s1 - mx), 0.0)
    e2 = jnp.where(k2, jnp.exp(s2 - mx), 0.0)
    den = e0 + e1 + e2
    inv = jnp.where(den > 0, 1.0 / den, 0.0)
    a0 = e0 * inv
    a1 = e1 * inv
    a2 = e2 * inv
    h = m0 * a0 + m1 * a1 + m2 * a2 + bias_ref[...] + sl_ref[...]
    out = _gelu(h)
    final_ref[...] = _layernorm_in(out, gf_ref[...], bf_ref[...])
    attn_ref[...] = jnp.concatenate([a0, a1, a2], axis=1)


def _post_call(part, sl, av2, bias2, gf, bf):
    B = 1000
    n_blk = _N // B
    full = lambda i: (0, 0)
    row_spec = pl.BlockSpec((B, _D), lambda i: (i, 0))
    return pl.pallas_call(
        _post_body,
        grid=(n_blk,),
        in_specs=[
            pl.BlockSpec((_R, _NC, B, _D), lambda i: (0, 0, i, 0)),
            row_spec,
            pl.BlockSpec((1, _D), full),
            pl.BlockSpec((1, _D), full),
            pl.BlockSpec((1, _D), full),
            pl.BlockSpec((1, _D), full),
        ],
        out_specs=[row_spec, pl.BlockSpec((B, _R), lambda i: (i, 0))],
        out_shape=[jax.ShapeDtypeStruct((_N, _D), jnp.float32),
                   jax.ShapeDtypeStruct((_N, _R), jnp.float32)],
    )(part, sl, av2, bias2, gf, bf)


def _tgt_body(final_hbm, idx_hbm, out_hbm, idxv, rowsv, sem):
    wid = lax.axis_index("s") * _NC + lax.axis_index("c")
    base = wid * 32
    pltpu.sync_copy(idx_hbm.at[pl.ds(base, 32)], idxv)
    pltpu.async_copy(final_hbm.at[idxv], rowsv, sem).wait()
    pltpu.sync_copy(rowsv, out_hbm.at[pl.ds(base, 32)])


def _tgt_call(final_full, tgt):
    f = pl.kernel(
        _tgt_body,
        out_type=jax.ShapeDtypeStruct((1024, _D), jnp.float32),
        mesh=plsc.VectorSubcoreMesh(core_axis_name="c", subcore_axis_name="s",
                                    num_cores=_NC),
        scratch_types=[
            pltpu.VMEM((32,), jnp.int32),
            pltpu.VMEM((32, _D), jnp.float32),
            pltpu.SemaphoreType.DMA,
        ],
    )
    return f(final_full, tgt)


def kernel(target_node_indices, emb, W_pre, b_pre, ln1_g, ln1_b, Wr, attn_vec,
           Ws, bs, bias, lnf_g, lnf_b, rows, cols, vals):
    bpre2 = b_pre.reshape(1, _D)
    g12 = ln1_g.reshape(1, _D)
    b12 = ln1_b.reshape(1, _D)
    bs2 = bs.reshape(1, _D)
    bias2 = bias.reshape(1, _D)
    gf2 = lnf_g.reshape(1, _D)
    bf2 = lnf_b.reshape(1, _D)
    av2 = attn_vec.reshape(1, _D)

    wcat = jnp.concatenate([Wr[0], Wr[1], Wr[2], Ws], axis=0)
    xr0, xr1, xr2, sl = _pre_call(emb, W_pre, bpre2, g12, b12, wcat, bs2)

    nw = _NC * _NT
    real = _E // nw
    padw = _EPT - real
    r3 = rows.reshape(_R, nw, real)
    c3 = cols.reshape(_R, nw, real)
    v3 = vals.reshape(_R, nw, real)
    pidx = (jnp.arange(nw, dtype=jnp.int32)[None, :, None] * padw
            + jnp.arange(padw, dtype=jnp.int32)[None, None, :])
    pidx = jnp.broadcast_to(pidx, (_R, nw, padw))
    rows_f = jnp.concatenate([r3, pidx], axis=2).reshape(_R * _EPAD)
    cols_f = jnp.concatenate([c3, pidx], axis=2).reshape(_R * _EPAD)
    vals_f = jnp.concatenate(
        [v3, jnp.zeros((_R, nw, padw), jnp.float32)], axis=2).reshape(_R * _EPAD)

    part = _spmm_call(xr0, xr1, xr2, rows_f, cols_f, vals_f)

    final_full, attn = _post_call(part, sl, av2, bias2, gf2, bf2)
    final = _tgt_call(final_full, target_node_indices)
    return final, attn, emb


# speedup vs baseline: 7.7453x; 1.0156x over previous
import functools

import jax
import jax.numpy as jnp
import numpy as np
from jax import lax
from jax.experimental import pallas as pl
from jax.experimental.pallas import tpu as pltpu
from jax.experimental.pallas import tpu_sc as plsc

_N = 10000
_D = 128
_R = 3
_E = 160000

_NC = 2
_NT = 16
_K = 112
_CPT = 45
_EPT = _K * _CPT
_EPAD = _EPT * _NC * _NT
_SQRT1_2 = np.float32(0.7071067811865476)


def _gelu(h):
    return 0.5 * h * (1.0 + lax.erf(h * _SQRT1_2))


def _layernorm_in(h, g, b):
    mu = jnp.mean(h, axis=1, keepdims=True)
    var = jnp.mean((h - mu) ** 2, axis=1, keepdims=True)
    return (h - mu) * lax.rsqrt(var + np.float32(1e-5)) * g + b


def _pre_body(emb_ref, wpre_ref, bpre_ref, g1_ref, b1_ref, wcat_ref,
              bs_ref, xr0_ref, xr1_ref, xr2_ref, sl_ref):
    e = emb_ref[...]
    h = lax.dot_general(e, wpre_ref[...], (((1,), (1,)), ((), ())),
                        preferred_element_type=jnp.float32) + bpre_ref[...]
    h = _layernorm_in(h, g1_ref[...], b1_ref[...])
    x = _gelu(h)
    h2 = lax.dot_general(x, wcat_ref[...], (((1,), (1,)), ((), ())),
                         preferred_element_type=jnp.float32)
    xr0_ref[...] = h2[:, :_D]
    xr1_ref[...] = h2[:, _D:2 * _D]
    xr2_ref[...] = h2[:, 2 * _D:3 * _D]
    sl_ref[...] = h2[:, 3 * _D:] + bs_ref[...]


def _pre_call(emb, wpre, bpre, g1, b1, wcat, bs2):
    B = 1000
    n_blk = _N // B
    full = lambda i: (0, 0)
    row_spec = pl.BlockSpec((B, _D), lambda i: (i, 0))
    return pl.pallas_call(
        _pre_body,
        grid=(n_blk,),
        in_specs=[
            row_spec,
            pl.BlockSpec((_D, _D), full),
            pl.BlockSpec((1, _D), full),
            pl.BlockSpec((1, _D), full),
            pl.BlockSpec((1, _D), full),
            pl.BlockSpec((4 * _D, _D), full),
            pl.BlockSpec((1, _D), full),
        ],
        out_specs=[row_spec, row_spec, row_spec, row_spec],
        out_shape=[jax.ShapeDtypeStruct((_N, _D), jnp.float32)] * 4,
    )(emb, wpre, bpre, g1, b1, wcat, bs2)


def _scale_chunk(buf, valbuf):
    @plsc.parallel_loop(0, _K // 16, unroll=2)
    def _grp(g):
        v16 = valbuf[pl.ds(g * 16, 16)]
        for j in range(16):
            vj = v16[j]
            k = g * 16 + j
            for t in range(_D // 16):
                buf[k, pl.ds(t * 16, 16)] = buf[k, pl.ds(t * 16, 16)] * vj


_NTRIP = _CPT // 3


def _spmm_body(xr0_hbm, xr1_hbm, xr2_hbm,
               rows_hbm, cols_hbm, vals_hbm,
               out_hbm,
               buf0, buf1, buf2, col0, col1, col2, row0b, row1b, row2b,
               val0, val1, val2, acc,
               gsem0, gsem1, gsem2, esem0, esem1, esem2, ssem0, ssem1, ssem2):
    c = lax.axis_index("c")
    s = lax.axis_index("s")
    xr_list = [xr0_hbm, xr1_hbm, xr2_hbm]
    bufs = [buf0, buf1, buf2]
    colb = [col0, col1, col2]
    rowb = [row0b, row1b, row2b]
    valb = [val0, val1, val2]
    gsems = [gsem0, gsem1, gsem2]
    esems = [esem0, esem1, esem2]
    ssems = [ssem0, ssem1, ssem2]
    row0 = s * 624

    for r in range(_R):
        base = r * _EPAD + (c * _NT + s) * _EPT
        xr = xr_list[r]

        def _estart(i, b):
            off = base + i * _K
            pltpu.async_copy(cols_hbm.at[pl.ds(off, _K)], colb[b], esems[b])
            pltpu.async_copy(rows_hbm.at[pl.ds(off, _K)], rowb[b], esems[b])
            pltpu.async_copy(vals_hbm.at[pl.ds(off, _K)], valb[b], esems[b])

        _estart(0, 0)
        _estart(1, 1)

        @pl.loop(0, _K)
        def _zero(k):
            for t in range(_D // 16):
                buf0[k, pl.ds(t * 16, 16)] = jnp.zeros((16,), jnp.float32)

        for z in range(5):
            pltpu.sync_copy(buf0.at[pl.ds(0, _K), :],
                            acc.at[pl.ds(row0 + z * _K, _K), :])
        pltpu.sync_copy(buf0.at[pl.ds(0, 64), :], acc.at[pl.ds(row0 + 560, 64), :])

        @pl.when(s == _NT - 1)
        def _zero_tail():
            pltpu.sync_copy(buf0.at[pl.ds(0, 16), :], acc.at[pl.ds(9984, 16), :])

        def _ewait_gstart(b):
            pltpu.make_async_copy(cols_hbm.at[pl.ds(base, _K)], colb[b], esems[b]).wait()
            pltpu.make_async_copy(rows_hbm.at[pl.ds(base, _K)], rowb[b], esems[b]).wait()
            pltpu.make_async_copy(vals_hbm.at[pl.ds(base, _K)], valb[b], esems[b]).wait()
            pltpu.async_copy(xr.at[colb[b]], bufs[b], gsems[b])

        def _gwait(b):
            pltpu.make_async_copy(xr.at[colb[b]], bufs[b], gsems[b]).wait()

        def _sstart(b):
            pltpu.async_copy(bufs[b], acc.at[rowb[b]], ssems[b], add=True)

        def _swait(b):
            pltpu.make_async_copy(bufs[b], acc.at[rowb[b]], ssems[b]).wait()

        _ewait_gstart(0)
        plsc.subcore_barrier()

        @pl.loop(0, _NTRIP)
        def _triple(j):
            for t in range(3):
                b = t
                b1 = (t + 1) % 3
                bp = (t + 2) % 3
                if t < 2:
                    _ewait_gstart(b1)
                else:
                    @pl.when(j < _NTRIP - 1)
                    def _head():
                        _ewait_gstart(b1)
                _gwait(b)
                _scale_chunk(bufs[b], valb[b])
                _sstart(b)
                if t == 0:
                    @pl.when(j > 0)
                    def _t0w():
                        _swait(bp)
                    _estart(3 * j + 2, bp)
                else:
                    @pl.when(j < _NTRIP - 1)
                    def _tail():
                        _swait(bp)
                        _estart(3 * j + t + 2, bp)

        _swait(0)
        _swait(1)
        _swait(2)

        plsc.subcore_barrier()
        pltpu.sync_copy(acc.at[pl.ds(row0, 624), :],
                        out_hbm.at[r, c, pl.ds(row0, 624), :])

        @pl.when(s == _NT - 1)
        def _wb_tail():
            pltpu.sync_copy(acc.at[pl.ds(9984, 16), :],
                            out_hbm.at[r, c, pl.ds(9984, 16), :])


def _spmm_call(xr0, xr1, xr2, rows_f, cols_f, vals_f):
    mesh = plsc.VectorSubcoreMesh(core_axis_name="c", subcore_axis_name="s",
                                  num_cores=_NC)
    f = pl.kernel(
        _spmm_body,
        out_type=jax.ShapeDtypeStruct((_R, _NC, _N, _D), jnp.float32),
        mesh=mesh,
        scratch_types=(
            [pltpu.VMEM((_K, _D), jnp.float32)] * 3
            + [pltpu.VMEM((_K,), jnp.int32)] * 6
            + [pltpu.VMEM((_K,), jnp.float32)] * 3
            + [pltpu.VMEM_SHARED((_N, _D), jnp.float32)]
            + [pltpu.SemaphoreType.DMA] * 9
        ),
    )
    return f(xr0, xr1, xr2, rows_f, cols_f, vals_f)


def _post_body(part_ref, sl_ref, av_ref, bias_ref, gf_ref, bf_ref,
               final_ref, attn_ref):
    m0 = part_ref[0, 0] + part_ref[0, 1]
    m1 = part_ref[1, 0] + part_ref[1, 1]
    m2 = part_ref[2, 0] + part_ref[2, 1]
    av = av_ref[...]
    s0 = jnp.sum(m0 * av, axis=1, keepdims=True)
    s1 = jnp.sum(m1 * av, axis=1, keepdims=True)
    s2 = jnp.sum(m2 * av, axis=1, keepdims=True)
    k0 = jnp.any(m0 != 0, axis=1, keepdims=True)
    k1 = jnp.any(m1 != 0, axis=1, keepdims=True)
    k2 = jnp.any(m2 != 0, axis=1, keepdims=True)
    neg = np.float32(-1e30)
    s0 = jnp.where(k0, s0, neg)
    s1 = jnp.where(k1, s1, neg)
    s2 = jnp.where(k2, s2, neg)
    mx = jnp.maximum(jnp.maximum(s0, s1), s2)
    e0 = jnp.where(k0, jnp.exp(s0 - mx), 0.0)
    e1 = jnp.where(k1, jnp.exp(s1 - mx), 0.0)
    e2 = jnp.where(k2, jnp.exp(s2 - mx), 0.0)
    den = e0 + e1 + e2
    inv = jnp.where(den > 0, 1.0 / den, 0.0)
    a0 = e0 * inv
    a1 = e1 * inv
    a2 = e2 * inv
    h = m0 * a0 + m1 * a1 + m2 * a2 + bias_ref[...] + sl_ref[...]
    out = _gelu(h)
    final_ref[...] = _layernorm_in(out, gf_ref[...], bf_ref[...])
    attn_ref[...] = jnp.concatenate([a0, a1, a2], axis=1)


def _post_call(part, sl, av2, bias2, gf, bf):
    B = 1000
    n_blk = _N // B
    full = lambda i: (0, 0)
    row_spec = pl.BlockSpec((B, _D), lambda i: (i, 0))
    return pl.pallas_call(
        _post_body,
        grid=(n_blk,),
        in_specs=[
            pl.BlockSpec((_R, _NC, B, _D), lambda i: (0, 0, i, 0)),
            row_spec,
            pl.BlockSpec((1, _D), full),
            pl.BlockSpec((1, _D), full),
            pl.BlockSpec((1, _D), full),
            pl.BlockSpec((1, _D), full),
        ],
        out_specs=[row_spec, pl.BlockSpec((B, _R), lambda i: (i, 0))],
        out_shape=[jax.ShapeDtypeStruct((_N, _D), jnp.float32),
                   jax.ShapeDtypeStruct((_N, _R), jnp.float32)],
    )(part, sl, av2, bias2, gf, bf)


def _tgt_body(final_hbm, idx_hbm, out_hbm, idxv, rowsv, sem):
    wid = lax.axis_index("s") * _NC + lax.axis_index("c")
    base = wid * 32
    pltpu.sync_copy(idx_hbm.at[pl.ds(base, 32)], idxv)
    pltpu.async_copy(final_hbm.at[idxv], rowsv, sem).wait()
    pltpu.sync_copy(rowsv, out_hbm.at[pl.ds(base, 32)])


def _tgt_call(final_full, tgt):
    f = pl.kernel(
        _tgt_body,
        out_type=jax.ShapeDtypeStruct((1024, _D), jnp.float32),
        mesh=plsc.VectorSubcoreMesh(core_axis_name="c", subcore_axis_name="s",
                                    num_cores=_NC),
        scratch_types=[
            pltpu.VMEM((32,), jnp.int32),
            pltpu.VMEM((32, _D), jnp.float32),
            pltpu.SemaphoreType.DMA,
        ],
    )
    return f(final_full, tgt)


def kernel(target_node_indices, emb, W_pre, b_pre, ln1_g, ln1_b, Wr, attn_vec,
           Ws, bs, bias, lnf_g, lnf_b, rows, cols, vals):
    bpre2 = b_pre.reshape(1, _D)
    g12 = ln1_g.reshape(1, _D)
    b12 = ln1_b.reshape(1, _D)
    bs2 = bs.reshape(1, _D)
    bias2 = bias.reshape(1, _D)
    gf2 = lnf_g.reshape(1, _D)
    bf2 = lnf_b.reshape(1, _D)
    av2 = attn_vec.reshape(1, _D)

    wcat = jnp.concatenate([Wr[0], Wr[1], Wr[2], Ws], axis=0)
    xr0, xr1, xr2, sl = _pre_call(emb, W_pre, bpre2, g12, b12, wcat, bs2)

    nw = _NC * _NT
    real = _E // nw
    padw = _EPT - real
    r3 = rows.reshape(_R, nw, real)
    c3 = cols.reshape(_R, nw, real)
    v3 = vals.reshape(_R, nw, real)
    pidx = (jnp.arange(nw, dtype=jnp.int32)[None, :, None] * padw
            + jnp.arange(padw, dtype=jnp.int32)[None, None, :])
    pidx = jnp.broadcast_to(pidx, (_R, nw, padw))
    rows_f = jnp.concatenate([r3, pidx], axis=2).reshape(_R * _EPAD)
    cols_f = jnp.concatenate([c3, pidx], axis=2).reshape(_R * _EPAD)
    vals_f = jnp.concatenate(
        [v3, jnp.zeros((_R, nw, padw), jnp.float32)], axis=2).reshape(_R * _EPAD)

    part = _spmm_call(xr0, xr1, xr2, rows_f, cols_f, vals_f)

    final_full, attn = _post_call(part, sl, av2, bias2, gf2, bf2)
    final = _tgt_call(final_full, target_node_indices)
    return final, attn, emb


# speedup vs baseline: 7.8801x; 1.0174x over previous
import functools

import jax
import jax.numpy as jnp
import numpy as np
from jax import lax
from jax.experimental import pallas as pl
from jax.experimental.pallas import tpu as pltpu
from jax.experimental.pallas import tpu_sc as plsc

_N = 10000
_D = 128
_R = 3
_E = 160000

_NC = 2
_NT = 16
_K = 112
_CPT = 45
_EPT = _K * _CPT
_EPAD = _EPT * _NC * _NT
_SQRT1_2 = np.float32(0.7071067811865476)


def _gelu(h):
    return 0.5 * h * (1.0 + lax.erf(h * _SQRT1_2))


def _layernorm_in(h, g, b):
    mu = jnp.mean(h, axis=1, keepdims=True)
    var = jnp.mean((h - mu) ** 2, axis=1, keepdims=True)
    return (h - mu) * lax.rsqrt(var + np.float32(1e-5)) * g + b


def _pre_body(emb_ref, wpre_ref, bpre_ref, g1_ref, b1_ref, wcat_ref,
              bs_ref, xr0_ref, xr1_ref, xr2_ref, sl_ref):
    e = emb_ref[...]
    h = lax.dot_general(e, wpre_ref[...], (((1,), (1,)), ((), ())),
                        preferred_element_type=jnp.float32) + bpre_ref[...]
    h = _layernorm_in(h, g1_ref[...], b1_ref[...])
    x = _gelu(h)
    h2 = lax.dot_general(x, wcat_ref[...], (((1,), (1,)), ((), ())),
                         preferred_element_type=jnp.float32)
    xr0_ref[...] = h2[:, :_D]
    xr1_ref[...] = h2[:, _D:2 * _D]
    xr2_ref[...] = h2[:, 2 * _D:3 * _D]
    sl_ref[...] = h2[:, 3 * _D:] + bs_ref[...]


def _pre_call(emb, wpre, bpre, g1, b1, wcat, bs2):
    B = 2000
    n_blk = _N // B
    full = lambda i: (0, 0)
    row_spec = pl.BlockSpec((B, _D), lambda i: (i, 0))
    return pl.pallas_call(
        _pre_body,
        grid=(n_blk,),
        in_specs=[
            row_spec,
            pl.BlockSpec((_D, _D), full),
            pl.BlockSpec((1, _D), full),
            pl.BlockSpec((1, _D), full),
            pl.BlockSpec((1, _D), full),
            pl.BlockSpec((4 * _D, _D), full),
            pl.BlockSpec((1, _D), full),
        ],
        out_specs=[row_spec, row_spec, row_spec, row_spec],
        out_shape=[jax.ShapeDtypeStruct((_N, _D), jnp.float32)] * 4,
    )(emb, wpre, bpre, g1, b1, wcat, bs2)


def _scale_chunk(buf, valbuf):
    @plsc.parallel_loop(0, _K // 16, unroll=2)
    def _grp(g):
        v16 = valbuf[pl.ds(g * 16, 16)]
        for j in range(16):
            vj = v16[j]
            k = g * 16 + j
            for t in range(_D // 16):
                buf[k, pl.ds(t * 16, 16)] = buf[k, pl.ds(t * 16, 16)] * vj


_NTRIP = _CPT // 3


def _spmm_body(xr0_hbm, xr1_hbm, xr2_hbm,
               rows_hbm, cols_hbm, vals_hbm,
               out_hbm,
               buf0, buf1, buf2, col0, col1, col2, row0b, row1b, row2b,
               val0, val1, val2, acc,
               gsem0, gsem1, gsem2, esem0, esem1, esem2, ssem0, ssem1, ssem2):
    c = lax.axis_index("c")
    s = lax.axis_index("s")
    xr_list = [xr0_hbm, xr1_hbm, xr2_hbm]
    bufs = [buf0, buf1, buf2]
    colb = [col0, col1, col2]
    rowb = [row0b, row1b, row2b]
    valb = [val0, val1, val2]
    gsems = [gsem0, gsem1, gsem2]
    esems = [esem0, esem1, esem2]
    ssems = [ssem0, ssem1, ssem2]
    row0 = s * 624

    for r in range(_R):
        base = r * _EPAD + (c * _NT + s) * _EPT
        xr = xr_list[r]

        def _estart(i, b):
            off = base + i * _K
            pltpu.async_copy(cols_hbm.at[pl.ds(off, _K)], colb[b], esems[b])
            pltpu.async_copy(rows_hbm.at[pl.ds(off, _K)], rowb[b], esems[b])
            pltpu.async_copy(vals_hbm.at[pl.ds(off, _K)], valb[b], esems[b])

        _estart(0, 0)
        _estart(1, 1)

        @pl.loop(0, _K)
        def _zero(k):
            for t in range(_D // 16):
                buf0[k, pl.ds(t * 16, 16)] = jnp.zeros((16,), jnp.float32)

        for z in range(5):
            pltpu.sync_copy(buf0.at[pl.ds(0, _K), :],
                            acc.at[pl.ds(row0 + z * _K, _K), :])
        pltpu.sync_copy(buf0.at[pl.ds(0, 64), :], acc.at[pl.ds(row0 + 560, 64), :])

        @pl.when(s == _NT - 1)
        def _zero_tail():
            pltpu.sync_copy(buf0.at[pl.ds(0, 16), :], acc.at[pl.ds(9984, 16), :])

        def _ewait_gstart(b):
            pltpu.make_async_copy(cols_hbm.at[pl.ds(base, _K)], colb[b], esems[b]).wait()
            pltpu.make_async_copy(rows_hbm.at[pl.ds(base, _K)], rowb[b], esems[b]).wait()
            pltpu.make_async_copy(vals_hbm.at[pl.ds(base, _K)], valb[b], esems[b]).wait()
            pltpu.async_copy(xr.at[colb[b]], bufs[b], gsems[b])

        def _gwait(b):
            pltpu.make_async_copy(xr.at[colb[b]], bufs[b], gsems[b]).wait()

        def _sstart(b):
            pltpu.async_copy(bufs[b], acc.at[rowb[b]], ssems[b], add=True)

        def _swait(b):
            pltpu.make_async_copy(bufs[b], acc.at[rowb[b]], ssems[b]).wait()

        _ewait_gstart(0)
        plsc.subcore_barrier()

        @pl.loop(0, _NTRIP)
        def _triple(j):
            for t in range(3):
                b = t
                b1 = (t + 1) % 3
                bp = (t + 2) % 3
                if t < 2:
                    _ewait_gstart(b1)
                else:
                    @pl.when(j < _NTRIP - 1)
                    def _head():
                        _ewait_gstart(b1)
                _gwait(b)
                _scale_chunk(bufs[b], valb[b])
                _sstart(b)
                if t == 0:
                    @pl.when(j > 0)
                    def _t0w():
                        _swait(bp)
                    _estart(3 * j + 2, bp)
                else:
                    @pl.when(j < _NTRIP - 1)
                    def _tail():
                        _swait(bp)
                        _estart(3 * j + t + 2, bp)

        _swait(0)
        _swait(1)
        _swait(2)

        plsc.subcore_barrier()
        pltpu.sync_copy(acc.at[pl.ds(row0, 624), :],
                        out_hbm.at[r, c, pl.ds(row0, 624), :])

        @pl.when(s == _NT - 1)
        def _wb_tail():
            pltpu.sync_copy(acc.at[pl.ds(9984, 16), :],
                            out_hbm.at[r, c, pl.ds(9984, 16), :])


def _spmm_call(xr0, xr1, xr2, rows_f, cols_f, vals_f):
    mesh = plsc.VectorSubcoreMesh(core_axis_name="c", subcore_axis_name="s",
                                  num_cores=_NC)
    f = pl.kernel(
        _spmm_body,
        out_type=jax.ShapeDtypeStruct((_R, _NC, _N, _D), jnp.float32),
        mesh=mesh,
        scratch_types=(
            [pltpu.VMEM((_K, _D), jnp.float32)] * 3
            + [pltpu.VMEM((_K,), jnp.int32)] * 6
            + [pltpu.VMEM((_K,), jnp.float32)] * 3
            + [pltpu.VMEM_SHARED((_N, _D), jnp.float32)]
            + [pltpu.SemaphoreType.DMA] * 9
        ),
    )
    return f(xr0, xr1, xr2, rows_f, cols_f, vals_f)


def _post_body(part_ref, sl_ref, av_ref, bias_ref, gf_ref, bf_ref,
               final_ref, attn_ref):
    m0 = part_ref[0, 0] + part_ref[0, 1]
    m1 = part_ref[1, 0] + part_ref[1, 1]
    m2 = part_ref[2, 0] + part_ref[2, 1]
    av = av_ref[...]
    s0 = jnp.sum(m0 * av, axis=1, keepdims=True)
    s1 = jnp.sum(m1 * av, axis=1, keepdims=True)
    s2 = jnp.sum(m2 * av, axis=1, keepdims=True)
    k0 = jnp.any(m0 != 0, axis=1, keepdims=True)
    k1 = jnp.any(m1 != 0, axis=1, keepdims=True)
    k2 = jnp.any(m2 != 0, axis=1, keepdims=True)
    neg = np.float32(-1e30)
    s0 = jnp.where(k0, s0, neg)
    s1 = jnp.where(k1, s1, neg)
    s2 = jnp.where(k2, s2, neg)
    mx = jnp.maximum(jnp.maximum(s0, s1), s2)
    e0 = jnp.where(k0, jnp.exp(s0 - mx), 0.0)
    e1 = jnp.where(k1, jnp.exp(s1 - mx), 0.0)
    e2 = jnp.where(k2, jnp.exp(s2 - mx), 0.0)
    den = e0 + e1 + e2
    inv = jnp.where(den > 0, 1.0 / den, 0.0)
    a0 = e0 * inv
    a1 = e1 * inv
    a2 = e2 * inv
    h = m0 * a0 + m1 * a1 + m2 * a2 + bias_ref[...] + sl_ref[...]
    out = _gelu(h)
    final_ref[...] = _layernorm_in(out, gf_ref[...], bf_ref[...])
    attn_ref[...] = jnp.concatenate([a0, a1, a2], axis=1)


def _post_call(part, sl, av2, bias2, gf, bf):
    B = 2000
    n_blk = _N // B
    full = lambda i: (0, 0)
    row_spec = pl.BlockSpec((B, _D), lambda i: (i, 0))
    return pl.pallas_call(
        _post_body,
        grid=(n_blk,),
        in_specs=[
            pl.BlockSpec((_R, _NC, B, _D), lambda i: (0, 0, i, 0)),
            row_spec,
            pl.BlockSpec((1, _D), full),
            pl.BlockSpec((1, _D), full),
            pl.BlockSpec((1, _D), full),
            pl.BlockSpec((1, _D), full),
        ],
        out_specs=[row_spec, pl.BlockSpec((B, _R), lambda i: (i, 0))],
        out_shape=[jax.ShapeDtypeStruct((_N, _D), jnp.float32),
                   jax.ShapeDtypeStruct((_N, _R), jnp.float32)],
    )(part, sl, av2, bias2, gf, bf)


def _tgt_body(final_hbm, idx_hbm, out_hbm, idxv, rowsv, sem):
    wid = lax.axis_index("s") * _NC + lax.axis_index("c")
    base = wid * 32
    pltpu.sync_copy(idx_hbm.at[pl.ds(base, 32)], idxv)
    pltpu.async_copy(final_hbm.at[idxv], rowsv, sem).wait()
    pltpu.sync_copy(rowsv, out_hbm.at[pl.ds(base, 32)])


def _tgt_call(final_full, tgt):
    f = pl.kernel(
        _tgt_body,
        out_type=jax.ShapeDtypeStruct((1024, _D), jnp.float32),
        mesh=plsc.VectorSubcoreMesh(core_axis_name="c", subcore_axis_name="s",
                                    num_cores=_NC),
        scratch_types=[
            pltpu.VMEM((32,), jnp.int32),
            pltpu.VMEM((32, _D), jnp.float32),
            pltpu.SemaphoreType.DMA,
        ],
    )
    return f(final_full, tgt)


def kernel(target_node_indices, emb, W_pre, b_pre, ln1_g, ln1_b, Wr, attn_vec,
           Ws, bs, bias, lnf_g, lnf_b, rows, cols, vals):
    bpre2 = b_pre.reshape(1, _D)
    g12 = ln1_g.reshape(1, _D)
    b12 = ln1_b.reshape(1, _D)
    bs2 = bs.reshape(1, _D)
    bias2 = bias.reshape(1, _D)
    gf2 = lnf_g.reshape(1, _D)
    bf2 = lnf_b.reshape(1, _D)
    av2 = attn_vec.reshape(1, _D)

    wcat = jnp.concatenate([Wr[0], Wr[1], Wr[2], Ws], axis=0)
    xr0, xr1, xr2, sl = _pre_call(emb, W_pre, bpre2, g12, b12, wcat, bs2)

    nw = _NC * _NT
    real = _E // nw
    padw = _EPT - real
    r3 = rows.reshape(_R, nw, real)
    c3 = cols.reshape(_R, nw, real)
    v3 = vals.reshape(_R, nw, real)
    pidx = (jnp.arange(nw, dtype=jnp.int32)[None, :, None] * padw
            + jnp.arange(padw, dtype=jnp.int32)[None, None, :])
    pidx = jnp.broadcast_to(pidx, (_R, nw, padw))
    rows_f = jnp.concatenate([r3, pidx], axis=2).reshape(_R * _EPAD)
    cols_f = jnp.concatenate([c3, pidx], axis=2).reshape(_R * _EPAD)
    vals_f = jnp.concatenate(
        [v3, jnp.zeros((_R, nw, padw), jnp.float32)], axis=2).reshape(_R * _EPAD)

    part = _spmm_call(xr0, xr1, xr2, rows_f, cols_f, vals_f)

    final_full, attn = _post_call(part, sl, av2, bias2, gf2, bf2)
    final = _tgt_call(final_full, target_node_indices)
    return final, attn, emb
